# baseline (speedup 1.0000x reference)
import functools

import jax
import jax.numpy as jnp
from jax import lax
from jax.experimental import pallas as pl
from jax.experimental.pallas import tpu as pltpu
from jax.experimental.pallas import tpu_sc as plsc

N = 10000
E = 320000
D = 128
DH = D // 2
NC, NS, L = 2, 16, 16
NW = NC * NS
EPT = E // NW
EPA = E // NS
RPT = 632
NPAD = RPT * NS
K = 80
NCHB = EPA // K
NBUF = 3

_MESH = plsc.VectorSubcoreMesh(core_axis_name="c", subcore_axis_name="s")
_PREC = lax.Precision.HIGHEST


def _worker(base_len):
    cid = lax.axis_index("c")
    sid = lax.axis_index("s")
    wid = sid * NC + cid
    return cid, sid, pl.multiple_of(wid * base_len, 8)


def _zero_shared_rows(z_v, acc_sh, sid):
    nz = z_v.shape[0]
    zero = jnp.zeros((L,), jnp.float32)

    if len(z_v.shape) == 1:
        @pl.loop(0, nz, step=L)
        def _(i):
            z_v[pl.ds(i, L)] = zero
    else:
        @pl.loop(0, nz)
        def _(j):
            for q in range(z_v.shape[1] // L):
                z_v[j, pl.ds(q * L, L)] = zero

    start = pl.multiple_of(sid * RPT, 8)
    off = 0
    while off < RPT:
        step = min(nz, RPT - off)
        pltpu.sync_copy(z_v.at[pl.ds(0, step)], acc_sh.at[pl.ds(start + off, step)])
        off += step


@functools.partial(
    pl.kernel,
    out_type=jax.ShapeDtypeStruct((NC * NPAD,), jnp.float32),
    mesh=_MESH,
    scratch_types=[
        pltpu.VMEM_SHARED((NPAD,), jnp.float32),
        pltpu.VMEM((EPT,), jnp.float32),
        pltpu.VMEM((EPT,), jnp.int32),
        pltpu.VMEM((RPT,), jnp.float32),
    ],
)
def _sc_deg(c_hbm, w_hbm, out_hbm, acc_sh, w_v, c_v, z_v):
    cid, sid, base = _worker(EPT)
    _zero_shared_rows(z_v, acc_sh, sid)
    plsc.subcore_barrier()
    pltpu.sync_copy(w_hbm.at[pl.ds(base, EPT)], w_v)
    pltpu.sync_copy(c_hbm.at[pl.ds(base, EPT)], c_v)
    pltpu.sync_copy(w_v, acc_sh.at[c_v], add=True)
    plsc.subcore_barrier()
    s = pl.multiple_of(sid * RPT, 8)
    d0 = pl.multiple_of(cid * NPAD + sid * RPT, 8)
    pltpu.sync_copy(acc_sh.at[pl.ds(s, RPT)], z_v)
    pltpu.sync_copy(z_v, out_hbm.at[pl.ds(d0, RPT)])


@functools.partial(
    pl.kernel,
    out_type=jax.ShapeDtypeStruct((NC * NPAD, DH), jnp.float32),
    mesh=_MESH,
    compiler_params=pltpu.CompilerParams(use_tc_tiling_on_sc=False),
    scratch_types=[
        pltpu.VMEM_SHARED((NPAD, DH), jnp.float32),
        pltpu.VMEM((K, DH), jnp.float32),
        pltpu.VMEM((K, DH), jnp.float32),
        pltpu.VMEM((K, DH), jnp.float32),
        pltpu.VMEM((EPA,), jnp.int32),
        pltpu.VMEM((EPA,), jnp.float32),
        pltpu.VMEM((K,), jnp.int32),
        pltpu.VMEM((K,), jnp.int32),
        pltpu.VMEM((K,), jnp.int32),
        pltpu.VMEM((K,), jnp.int32),
        pltpu.VMEM((K,), jnp.int32),
        pltpu.VMEM((K,), jnp.int32),
        pltpu.SemaphoreType.DMA,
        pltpu.SemaphoreType.DMA,
        pltpu.SemaphoreType.DMA,
        pltpu.SemaphoreType.DMA,
        pltpu.SemaphoreType.DMA,
        pltpu.SemaphoreType.DMA,
    ],
)
def _sc_agg(h_hbm, rc_hbm, w_hbm, out_hbm, acc_sh,
            rows0, rows1, rows2, rc_all, w_all, r0, r1, r2, c0, c1, c2,
            sg0, sg1, sg2, ss0, ss1, ss2):
    cid = lax.axis_index("c")
    sid = lax.axis_index("s")
    base = pl.multiple_of(sid * EPA, 8)
    cid_n = cid * N

    _zero_shared_rows(rows0, acc_sh, sid)
    pltpu.sync_copy(rc_hbm.at[pl.ds(base, EPA)], rc_all)
    pltpu.sync_copy(w_hbm.at[pl.ds(base, EPA)], w_all)
    plsc.subcore_barrier()

    bufs = (
        (rows0, r0, c0, sg0, ss0),
        (rows1, r1, c1, sg1, ss1),
        (rows2, r2, c2, sg2, ss2),
    )

    def start(k, buf):
        rows, r_v, c_v, sg, ss = buf
        off = pl.multiple_of(k * K, 8)
        for j in range(0, K, L):
            rc = rc_all[pl.ds(off + j, L)]
            r_v[pl.ds(j, L)] = lax.shift_right_logical(rc, 16) + cid_n
            c_v[pl.ds(j, L)] = lax.bitwise_and(rc, jnp.int32(0xFFFF))
        pltpu.async_copy(h_hbm.at[r_v], rows, sg)

    def wait_scatter(buf):
        rows, r_v, c_v, sg, ss = buf
        pltpu.make_async_copy(rows, acc_sh.at[c_v], ss).wait()

    def body(k, buf):
        rows, r_v, c_v, sg, ss = buf
        pltpu.make_async_copy(h_hbm.at[r_v], rows, sg).wait()
        woff = pl.multiple_of(k * K, 8)

        @pl.loop(0, K, step=L)
        def _(j0):
            wv = w_all[pl.ds(woff + j0, L)]
            for t in range(L):
                s = wv[t]
                for q in range(DH // L):
                    rows[j0 + t, pl.ds(q * L, L)] = rows[j0 + t, pl.ds(q * L, L)] * s

        pltpu.async_copy(rows, acc_sh.at[c_v], ss, add=True)

    start(0, bufs[0])

    @pl.loop(0, NCHB - 1, step=NBUF)
    def _(kk):
        for b in range(NBUF):
            k = kk + b
            bn = (b + 1) % NBUF

            @pl.when(k >= 2)
            def _():
                wait_scatter(bufs[bn])

            @pl.when(k + 1 < NCHB)
            def _():
                start(k + 1, bufs[bn])

            body(k, bufs[b])

    k = NCHB - 1
    wait_scatter(bufs[(k - 2) % NBUF])
    body(k, bufs[k % NBUF])
    wait_scatter(bufs[(k - 1) % NBUF])
    wait_scatter(bufs[k % NBUF])

    plsc.subcore_barrier()
    s0 = pl.multiple_of(sid * RPT, 8)
    d0 = pl.multiple_of(cid * NPAD + sid * RPT, 8)
    off = 0
    while off < RPT:
        step = min(K, RPT - off)
        pltpu.sync_copy(acc_sh.at[pl.ds(s0 + off, step)], rows0.at[pl.ds(0, step)])
        pltpu.sync_copy(rows0.at[pl.ds(0, step)], out_hbm.at[pl.ds(d0 + off, step)])
        off += step


@functools.partial(
    pl.kernel,
    out_type=jax.ShapeDtypeStruct((NC * NPAD,), jnp.float32),
    mesh=_MESH,
    scratch_types=[
        pltpu.VMEM_SHARED((NPAD,), jnp.float32),
        pltpu.VMEM((EPT,), jnp.int32),
        pltpu.VMEM((EPT,), jnp.int32),
        pltpu.VMEM((EPT,), jnp.float32),
        pltpu.VMEM((EPT,), jnp.float32),
        pltpu.VMEM((RPT,), jnp.float32),
        pltpu.SemaphoreType.DMA,
    ],
)
def _sc_agg1(g_hbm, r_hbm, c_hbm, w_hbm, out_hbm, acc_sh,
             r_v, c_v, w_v, g_v, z_v, sem):
    cid, sid, base = _worker(EPT)
    _zero_shared_rows(z_v, acc_sh, sid)
    plsc.subcore_barrier()
    pltpu.sync_copy(r_hbm.at[pl.ds(base, EPT)], r_v)
    pltpu.sync_copy(c_hbm.at[pl.ds(base, EPT)], c_v)
    pltpu.sync_copy(w_hbm.at[pl.ds(base, EPT)], w_v)
    pltpu.async_copy(g_hbm.at[r_v], g_v, sem).wait()

    @pl.loop(0, EPT, step=L)
    def _(i):
        g_v[pl.ds(i, L)] = g_v[pl.ds(i, L)] * w_v[pl.ds(i, L)]

    pltpu.sync_copy(g_v, acc_sh.at[c_v], add=True)
    plsc.subcore_barrier()
    s = pl.multiple_of(sid * RPT, 8)
    d0 = pl.multiple_of(cid * NPAD + sid * RPT, 8)
    pltpu.sync_copy(acc_sh.at[pl.ds(s, RPT)], z_v)
    pltpu.sync_copy(z_v, out_hbm.at[pl.ds(d0, RPT)])


_BLK = 1000
_GRID = N // _BLK


def _dis_body(deg_ref, dis_ref):
    d = deg_ref[0, :] + deg_ref[1, :] + 1.0
    dis_ref[...] = lax.rsqrt(d)


def _tc_dis(deg_parts):
    return pl.pallas_call(
        _dis_body,
        out_shape=jax.ShapeDtypeStruct((NPAD,), jnp.float32),
    )(deg_parts)


def _split(h):
    return jnp.stack([h[:, :DH], h[:, DH:]], axis=0)


def _cat(s3):
    return jnp.concatenate([s3[0], s3[1]], axis=1)


def _mm2_body(x_ref, wa_ref, wb_ref, dis_ref, hp_ref, hb_ref):
    x = x_ref[...]
    d = dis_ref[...]
    ha = lax.dot_general(x, wa_ref[...], (((1,), (1,)), ((), ())),
                         precision=_PREC, preferred_element_type=jnp.float32)
    hp_ref[...] = _split(d * ha)
    hb_ref[...] = lax.dot_general(x, wb_ref[...], (((1,), (1,)), ((), ())),
                                  precision=_PREC,
                                  preferred_element_type=jnp.float32)


def _tc_mm2(x, wa, wb, dis):
    return pl.pallas_call(
        _mm2_body,
        grid=(_GRID,),
        in_specs=[
            pl.BlockSpec((_BLK, D), lambda i: (i, 0)),
            pl.BlockSpec((D, D), lambda i: (0, 0)),
            pl.BlockSpec((D, D), lambda i: (0, 0)),
            pl.BlockSpec((_BLK, 1), lambda i: (i, 0)),
        ],
        out_specs=[
            pl.BlockSpec((2, _BLK, DH), lambda i: (0, i, 0)),
            pl.BlockSpec((_BLK, D), lambda i: (i, 0)),
        ],
        out_shape=[
            jax.ShapeDtypeStruct((2, N, DH), jnp.float32),
            jax.ShapeDtypeStruct((N, D), jnp.float32),
        ],
    )(x, wa, wb, dis)


def _post_mm_body(s_ref, h_ref, dis_ref, b_ref, w_ref, out_ref, *, split_out):
    d = dis_ref[...]
    z = jnp.maximum(d * (_cat(s_ref[...]) + _cat(h_ref[...])) + b_ref[...], 0.0)
    o = lax.dot_general(z, w_ref[...], (((1,), (1,)), ((), ())),
                        precision=_PREC, preferred_element_type=jnp.float32)
    if split_out:
        out_ref[...] = _split(d * o)
    else:
        out_ref[...] = d * o


def _tc_post_mm(s_parts, hp, dis, b, w, split_out, d_out):
    if split_out:
        out_spec = pl.BlockSpec((2, _BLK, DH), lambda i: (0, i, 0))
        out_shape = jax.ShapeDtypeStruct((2, N, DH), jnp.float32)
    else:
        out_spec = pl.BlockSpec((_BLK, d_out), lambda i: (i, 0))
        out_shape = jax.ShapeDtypeStruct((N, d_out), jnp.float32)
    return pl.pallas_call(
        functools.partial(_post_mm_body, split_out=split_out),
        grid=(_GRID,),
        in_specs=[
            pl.BlockSpec((2, _BLK, DH), lambda i: (0, i, 0)),
            pl.BlockSpec((2, _BLK, DH), lambda i: (0, i, 0)),
            pl.BlockSpec((_BLK, 1), lambda i: (i, 0)),
            pl.BlockSpec((1, D), lambda i: (0, 0)),
            pl.BlockSpec((d_out, D), lambda i: (0, 0)),
        ],
        out_specs=out_spec,
        out_shape=out_shape,
    )(s_parts, hp, dis, b, w)


def _xsol_body(s_ref, h_ref, dis_ref, b_ref, wol_ref, bol_ref, hga_ref,
               wg1b_ref, xsol_ref, hgp_ref):
    d = dis_ref[...]
    z = d * (_cat(s_ref[...]) + _cat(h_ref[...])) + b_ref[...]
    xsol = lax.dot_general(z, wol_ref[...], (((1,), (1,)), ((), ())),
                           precision=_PREC,
                           preferred_element_type=jnp.float32) + bol_ref[...]
    xsol_ref[...] = xsol
    hgp_ref[...] = _split(d * (hga_ref[...] + xsol[:, D - 1:D] * wg1b_ref[...]))


def _tc_xsol(s_parts, hp, dis, b, wol, bol, hga, wg1b):
    return pl.pallas_call(
        _xsol_body,
        grid=(_GRID,),
        in_specs=[
            pl.BlockSpec((2, _BLK, DH), lambda i: (0, i, 0)),
            pl.BlockSpec((2, _BLK, DH), lambda i: (0, i, 0)),
            pl.BlockSpec((_BLK, 1), lambda i: (i, 0)),
            pl.BlockSpec((1, D), lambda i: (0, 0)),
            pl.BlockSpec((D, D), lambda i: (0, 0)),
            pl.BlockSpec((1, D), lambda i: (0, 0)),
            pl.BlockSpec((_BLK, D), lambda i: (i, 0)),
            pl.BlockSpec((1, D), lambda i: (0, 0)),
        ],
        out_specs=[
            pl.BlockSpec((_BLK, D), lambda i: (i, 0)),
            pl.BlockSpec((2, _BLK, DH), lambda i: (0, i, 0)),
        ],
        out_shape=[
            jax.ShapeDtypeStruct((N, D), jnp.float32),
            jax.ShapeDtypeStruct((2, N, DH), jnp.float32),
        ],
    )(s_parts, hp, dis, b, wol, bol, hga, wg1b)


def _final_body(s_ref, h4p_ref, dis_ref, consts_ref, xsol_ref, xlast_ref,
                xnew_ref, gamma_ref):
    d = dis_ref[...]
    bg2 = consts_ref[0, 0]
    wgl = consts_ref[0, 1]
    bgl = consts_ref[0, 2]
    g = d * (s_ref[0] + s_ref[1] + h4p_ref[...]) + bg2
    gl = g * wgl + bgl
    gmin = jnp.min(gl)
    gmax = jnp.max(gl)
    gamma = (gl - gmin) / (gmax - gmin + 1e-12)
    gamma_ref[...] = gamma
    xsol = xsol_ref[...]
    xl = xlast_ref[...]
    upd = xl + gamma * (xsol[:, D - 1:D] - xl)
    col = lax.broadcasted_iota(jnp.int32, (N, D), 1)
    xnew_ref[...] = jnp.where(col == D - 1, upd, xsol)


def _tc_final(s_parts, h4p, dis, consts, xsol, xlast):
    return pl.pallas_call(
        _final_body,
        out_shape=[
            jax.ShapeDtypeStruct((N, D), jnp.float32),
            jax.ShapeDtypeStruct((N, 1), jnp.float32),
        ],
    )(s_parts, h4p, dis, consts, xsol, xlast)


def kernel(x, edge_index, edge_weights, Wo1, bo1, Wo2, bo2, Wol, bol,
           Wg1, bg1, Wg2, bg2, Wgl, bgl):
    row = edge_index[0].astype(jnp.int32)
    col = edge_index[1].astype(jnp.int32)
    ew = edge_weights.astype(jnp.float32)

    wg1a = Wg1[:, :D]
    wg1b = Wg1[:, D].reshape(1, D)
    rc = jnp.bitwise_or(jnp.left_shift(row, 16), col)

    deg_parts = _sc_deg(col, ew).reshape(NC, NPAD)
    dis1 = _tc_dis(deg_parts)[:N]
    dis = dis1.reshape(N, 1)

    h1p, hga = _tc_mm2(x, Wo1, wg1a, dis)

    s1 = _sc_agg(h1p.reshape(NC * N, DH), rc, ew).reshape(NC, NPAD, DH)
    h2p = _tc_post_mm(s1, h1p, dis, bo1.reshape(1, D), Wo2, True, D)

    s2 = _sc_agg(h2p.reshape(NC * N, DH), rc, ew).reshape(NC, NPAD, DH)
    xsol, hgp = _tc_xsol(s2, h2p, dis, bo2.reshape(1, D), Wol,
                         bol.reshape(1, D), hga, wg1b)

    s3 = _sc_agg(hgp.reshape(NC * N, DH), rc, ew).reshape(NC, NPAD, DH)
    h4p = _tc_post_mm(s3, hgp, dis, bg1.reshape(1, D), Wg2, False, 1)

    s4 = _sc_agg1(h4p.reshape(N), row, col, ew).reshape(NC, NPAD)[:, :N]

    consts = jnp.stack([bg2[0], Wgl[0, 0], bgl[0]]).reshape(1, 3)
    xnew, gamma = _tc_final(s4.reshape(NC, N, 1), h4p, dis, consts,
                            xsol, x[:, D - 1:D])
    return (xnew, gamma)


# speedup vs baseline: 1.2934x; 1.2934x over previous
import functools

import jax
import jax.numpy as jnp
from jax import lax
from jax.experimental import pallas as pl
from jax.experimental.pallas import tpu as pltpu
from jax.experimental.pallas import tpu_sc as plsc

N = 10000
E = 320000
D = 128
NC, NS, L = 2, 16, 16
NW = NC * NS
EPT = E // NW
RPT = 632
RPT_LAST = N - 15 * RPT
NPAD = RPT * NS
K = 64
NCH = EPT // K
TAIL = EPT - NCH * K
NBUF = 3
_LOOPED = (NCH - 2) - (NCH - 2) % NBUF

_MESH = plsc.VectorSubcoreMesh(core_axis_name="c", subcore_axis_name="s")
_PREC = lax.Precision.HIGHEST


def _worker(base_len):
    cid = lax.axis_index("c")
    sid = lax.axis_index("s")
    wid = sid * NC + cid
    return cid, sid, pl.multiple_of(wid * base_len, 8)


def _zero_rows(z_v, acc_sh, start, nrows):
    nz = z_v.shape[0]
    off = 0
    while off < nrows:
        step = min(nz, nrows - off)
        pltpu.sync_copy(z_v.at[pl.ds(0, step)], acc_sh.at[pl.ds(start + off, step)])
        off += step


def _fill_zero(z_v):
    nz = z_v.shape[0]
    zero = jnp.zeros((L,), jnp.float32)

    if len(z_v.shape) == 1:
        @pl.loop(0, nz, step=L)
        def _(i):
            z_v[pl.ds(i, L)] = zero
    else:
        @pl.loop(0, nz)
        def _(j):
            for q in range(z_v.shape[1] // L):
                z_v[j, pl.ds(q * L, L)] = zero


def _drain_rows(acc_sh, out_hbm, stage, start, dst, nrows):
    off = 0
    nz = stage.shape[0]
    while off < nrows:
        step = min(nz, nrows - off)
        pltpu.sync_copy(acc_sh.at[pl.ds(start + off, step)], stage.at[pl.ds(0, step)])
        pltpu.sync_copy(stage.at[pl.ds(0, step)], out_hbm.at[pl.ds(dst + off, step)])
        off += step


def _per_tile_rows(sid, fn):
    start = pl.multiple_of(sid * RPT, 8)

    @pl.when(sid < NS - 1)
    def _():
        fn(start, RPT)

    @pl.when(sid == NS - 1)
    def _():
        fn(start, RPT_LAST)


@functools.partial(
    pl.kernel,
    out_type=jax.ShapeDtypeStruct((NC * NPAD,), jnp.float32),
    mesh=_MESH,
    scratch_types=[
        pltpu.VMEM_SHARED((N,), jnp.float32),
        pltpu.VMEM((EPT,), jnp.float32),
        pltpu.VMEM((EPT,), jnp.int32),
        pltpu.VMEM((RPT,), jnp.float32),
    ],
)
def _sc_deg(c_hbm, w_hbm, out_hbm, acc_sh, w_v, c_v, z_v):
    cid, sid, base = _worker(EPT)
    _fill_zero(z_v)
    _per_tile_rows(sid, lambda s, n: _zero_rows(z_v, acc_sh, s, n))
    plsc.subcore_barrier()
    pltpu.sync_copy(w_hbm.at[pl.ds(base, EPT)], w_v)
    pltpu.sync_copy(c_hbm.at[pl.ds(base, EPT)], c_v)
    pltpu.sync_copy(w_v, acc_sh.at[c_v], add=True)
    plsc.subcore_barrier()
    d0 = pl.multiple_of(cid * NPAD + sid * RPT, 8)
    _per_tile_rows(sid, lambda s, n: _drain_rows(acc_sh, out_hbm, z_v, s, d0, n))


@functools.partial(
    pl.kernel,
    out_type=jax.ShapeDtypeStruct((NC * NPAD, D), jnp.float32),
    mesh=_MESH,
    scratch_types=[
        pltpu.VMEM_SHARED((N, D), jnp.float32),
        pltpu.VMEM((K, D), jnp.float32),
        pltpu.VMEM((K, D), jnp.float32),
        pltpu.VMEM((K, D), jnp.float32),
        pltpu.VMEM((TAIL, D), jnp.float32),
        pltpu.VMEM((EPT,), jnp.int32),
        pltpu.VMEM((EPT,), jnp.float32),
        pltpu.VMEM((K,), jnp.int32),
        pltpu.VMEM((K,), jnp.int32),
        pltpu.VMEM((K,), jnp.int32),
        pltpu.VMEM((TAIL,), jnp.int32),
        pltpu.SemaphoreType.DMA,
        pltpu.SemaphoreType.DMA,
        pltpu.SemaphoreType.DMA,
        pltpu.SemaphoreType.DMA,
        pltpu.SemaphoreType.DMA,
        pltpu.SemaphoreType.DMA,
    ],
)
def _sc_agg(h_hbm, rc_hbm, w_hbm, out_hbm, acc_sh,
            rows0, rows1, rows2, rows_t, rc_all, w_all, i0, i1, i2, i_t,
            sg0, sg1, sg2, ss0, ss1, ss2):
    cid, sid, base = _worker(EPT)
    _fill_zero(rows0)
    _per_tile_rows(sid, lambda s, n: _zero_rows(rows0, acc_sh, s, n))
    pltpu.sync_copy(rc_hbm.at[pl.ds(base, EPT)], rc_all)
    pltpu.sync_copy(w_hbm.at[pl.ds(base, EPT)], w_all)
    plsc.subcore_barrier()

    bufs = ((rows0, i0, sg0, ss0), (rows1, i1, sg1, ss1), (rows2, i2, sg2, ss2))

    def start(k, buf):
        rows, i_v, sg, ss = buf
        off = pl.multiple_of(k * K, 8)
        for j in range(0, K, L):
            rc = rc_all[pl.ds(off + j, L)]
            i_v[pl.ds(j, L)] = lax.shift_right_logical(rc, 16)
        pltpu.async_copy(h_hbm.at[i_v], rows, sg)

    def wait_scatter(buf):
        rows, i_v, sg, ss = buf
        pltpu.make_async_copy(rows, acc_sh.at[i_v], ss).wait()

    def body(k, buf):
        rows, i_v, sg, ss = buf
        pltpu.make_async_copy(h_hbm.at[i_v], rows, sg).wait()
        off = pl.multiple_of(k * K, 8)
        for j in range(0, K, L):
            rc = rc_all[pl.ds(off + j, L)]
            i_v[pl.ds(j, L)] = lax.bitwise_and(rc, jnp.int32(0xFFFF))

        @pl.loop(0, K, step=L)
        def _(j0):
            wv = w_all[pl.ds(off + j0, L)]
            for t in range(L):
                s = wv[t]
                for q in range(D // L):
                    rows[j0 + t, pl.ds(q * L, L)] = rows[j0 + t, pl.ds(q * L, L)] * s

        pltpu.async_copy(rows, acc_sh.at[i_v], ss, add=True)

    start(0, bufs[0])

    @pl.loop(0, _LOOPED, step=NBUF)
    def _(kk):
        for b in range(NBUF):
            k = kk + b
            bn = (b + 1) % NBUF

            @pl.when(k >= 2)
            def _():
                wait_scatter(bufs[bn])

            start(k + 1, bufs[bn])
            body(k, bufs[b])

    for k in range(_LOOPED, NCH):
        bn = (k + 1) % NBUF
        if k >= 2:
            wait_scatter(bufs[bn])
        if k + 1 < NCH:
            start(k + 1, bufs[bn])
        body(k, bufs[k % NBUF])
    wait_scatter(bufs[(NCH - 2) % NBUF])
    wait_scatter(bufs[(NCH - 1) % NBUF])

    toff = NCH * K
    rc = rc_all[pl.ds(toff, L)]
    i_t[pl.ds(0, L)] = lax.shift_right_logical(rc, 16)
    pltpu.async_copy(h_hbm.at[i_t], rows_t, sg0).wait()
    i_t[pl.ds(0, L)] = lax.bitwise_and(rc, jnp.int32(0xFFFF))
    wv = w_all[pl.ds(toff, L)]
    for t in range(L):
        s = wv[t]
        for q in range(D // L):
            rows_t[t, pl.ds(q * L, L)] = rows_t[t, pl.ds(q * L, L)] * s
    pltpu.sync_copy(rows_t, acc_sh.at[i_t], add=True)

    plsc.subcore_barrier()
    d0 = pl.multiple_of(cid * NPAD + sid * RPT, 8)
    _per_tile_rows(sid, lambda s, n: _drain_rows(acc_sh, out_hbm, rows0, s, d0, n))


@functools.partial(
    pl.kernel,
    out_type=jax.ShapeDtypeStruct((NC * NPAD,), jnp.float32),
    mesh=_MESH,
    scratch_types=[
        pltpu.VMEM_SHARED((N,), jnp.float32),
        pltpu.VMEM((EPT,), jnp.int32),
        pltpu.VMEM((EPT,), jnp.int32),
        pltpu.VMEM((EPT,), jnp.float32),
        pltpu.VMEM((EPT,), jnp.float32),
        pltpu.VMEM((RPT,), jnp.float32),
        pltpu.SemaphoreType.DMA,
    ],
)
def _sc_agg1(g_hbm, r_hbm, c_hbm, w_hbm, out_hbm, acc_sh,
             r_v, c_v, w_v, g_v, z_v, sem):
    cid, sid, base = _worker(EPT)
    _fill_zero(z_v)
    _per_tile_rows(sid, lambda s, n: _zero_rows(z_v, acc_sh, s, n))
    plsc.subcore_barrier()
    pltpu.sync_copy(r_hbm.at[pl.ds(base, EPT)], r_v)
    pltpu.sync_copy(c_hbm.at[pl.ds(base, EPT)], c_v)
    pltpu.sync_copy(w_hbm.at[pl.ds(base, EPT)], w_v)
    pltpu.async_copy(g_hbm.at[r_v], g_v, sem).wait()

    @pl.loop(0, EPT, step=L)
    def _(i):
        g_v[pl.ds(i, L)] = g_v[pl.ds(i, L)] * w_v[pl.ds(i, L)]

    pltpu.sync_copy(g_v, acc_sh.at[c_v], add=True)
    plsc.subcore_barrier()
    d0 = pl.multiple_of(cid * NPAD + sid * RPT, 8)
    _per_tile_rows(sid, lambda s, n: _drain_rows(acc_sh, out_hbm, z_v, s, d0, n))


_BLK = 1000
_GRID = N // _BLK


def _dis_body(deg_ref, dis_ref):
    d = deg_ref[0, :] + deg_ref[1, :] + 1.0
    dis_ref[...] = lax.rsqrt(d)


def _tc_dis(deg_parts):
    return pl.pallas_call(
        _dis_body,
        out_shape=jax.ShapeDtypeStruct((NPAD,), jnp.float32),
    )(deg_parts)


def _mm2_body(x_ref, wa_ref, wb_ref, dis_ref, hp_ref, hb_ref):
    x = x_ref[...]
    d = dis_ref[...]
    ha = lax.dot_general(x, wa_ref[...], (((1,), (1,)), ((), ())),
                         precision=_PREC, preferred_element_type=jnp.float32)
    hp_ref[...] = d * ha
    hb_ref[...] = lax.dot_general(x, wb_ref[...], (((1,), (1,)), ((), ())),
                                  precision=_PREC,
                                  preferred_element_type=jnp.float32)


def _tc_mm2(x, wa, wb, dis):
    return pl.pallas_call(
        _mm2_body,
        grid=(_GRID,),
        in_specs=[
            pl.BlockSpec((_BLK, D), lambda i: (i, 0)),
            pl.BlockSpec((D, D), lambda i: (0, 0)),
            pl.BlockSpec((D, D), lambda i: (0, 0)),
            pl.BlockSpec((_BLK, 1), lambda i: (i, 0)),
        ],
        out_specs=[
            pl.BlockSpec((_BLK, D), lambda i: (i, 0)),
            pl.BlockSpec((_BLK, D), lambda i: (i, 0)),
        ],
        out_shape=[
            jax.ShapeDtypeStruct((N, D), jnp.float32),
            jax.ShapeDtypeStruct((N, D), jnp.float32),
        ],
    )(x, wa, wb, dis)


def _post_mm_body(s_ref, h_ref, dis_ref, b_ref, w_ref, out_ref):
    d = dis_ref[...]
    z = jnp.maximum(d * (s_ref[0] + s_ref[1] + h_ref[...]) + b_ref[...], 0.0)
    o = lax.dot_general(z, w_ref[...], (((1,), (1,)), ((), ())),
                        precision=_PREC, preferred_element_type=jnp.float32)
    out_ref[...] = d * o


def _tc_post_mm(s_parts, hp, dis, b, w, d_out):
    return pl.pallas_call(
        _post_mm_body,
        grid=(_GRID,),
        in_specs=[
            pl.BlockSpec((2, _BLK, D), lambda i: (0, i, 0)),
            pl.BlockSpec((_BLK, D), lambda i: (i, 0)),
            pl.BlockSpec((_BLK, 1), lambda i: (i, 0)),
            pl.BlockSpec((1, D), lambda i: (0, 0)),
            pl.BlockSpec((d_out, D), lambda i: (0, 0)),
        ],
        out_specs=pl.BlockSpec((_BLK, d_out), lambda i: (i, 0)),
        out_shape=jax.ShapeDtypeStruct((N, d_out), jnp.float32),
    )(s_parts, hp, dis, b, w)


def _xsol_body(s_ref, h_ref, dis_ref, b_ref, wol_ref, bol_ref, hga_ref,
               wg1b_ref, xsol_ref, hgp_ref):
    d = dis_ref[...]
    z = d * (s_ref[0] + s_ref[1] + h_ref[...]) + b_ref[...]
    xsol = lax.dot_general(z, wol_ref[...], (((1,), (1,)), ((), ())),
                           precision=_PREC,
                           preferred_element_type=jnp.float32) + bol_ref[...]
    xsol_ref[...] = xsol
    hgp_ref[...] = d * (hga_ref[...] + xsol[:, D - 1:D] * wg1b_ref[...])


def _tc_xsol(s_parts, hp, dis, b, wol, bol, hga, wg1b):
    return pl.pallas_call(
        _xsol_body,
        grid=(_GRID,),
        in_specs=[
            pl.BlockSpec((2, _BLK, D), lambda i: (0, i, 0)),
            pl.BlockSpec((_BLK, D), lambda i: (i, 0)),
            pl.BlockSpec((_BLK, 1), lambda i: (i, 0)),
            pl.BlockSpec((1, D), lambda i: (0, 0)),
            pl.BlockSpec((D, D), lambda i: (0, 0)),
            pl.BlockSpec((1, D), lambda i: (0, 0)),
            pl.BlockSpec((_BLK, D), lambda i: (i, 0)),
            pl.BlockSpec((1, D), lambda i: (0, 0)),
        ],
        out_specs=[
            pl.BlockSpec((_BLK, D), lambda i: (i, 0)),
            pl.BlockSpec((_BLK, D), lambda i: (i, 0)),
        ],
        out_shape=[
            jax.ShapeDtypeStruct((N, D), jnp.float32),
            jax.ShapeDtypeStruct((N, D), jnp.float32),
        ],
    )(s_parts, hp, dis, b, wol, bol, hga, wg1b)


def _final_body(s_ref, h4p_ref, dis_ref, consts_ref, xsol_ref, xlast_ref,
                xnew_ref, gamma_ref):
    d = dis_ref[...]
    bg2 = consts_ref[0, 0]
    wgl = consts_ref[0, 1]
    bgl = consts_ref[0, 2]
    g = d * (s_ref[0] + s_ref[1] + h4p_ref[...]) + bg2
    gl = g * wgl + bgl
    gmin = jnp.min(gl)
    gmax = jnp.max(gl)
    gamma = (gl - gmin) / (gmax - gmin + 1e-12)
    gamma_ref[...] = gamma
    xsol = xsol_ref[...]
    xl = xlast_ref[...]
    upd = xl + gamma * (xsol[:, D - 1:D] - xl)
    col = lax.broadcasted_iota(jnp.int32, (N, D), 1)
    xnew_ref[...] = jnp.where(col == D - 1, upd, xsol)


def _tc_final(s_parts, h4p, dis, consts, xsol, xlast):
    return pl.pallas_call(
        _final_body,
        out_shape=[
            jax.ShapeDtypeStruct((N, D), jnp.float32),
            jax.ShapeDtypeStruct((N, 1), jnp.float32),
        ],
    )(s_parts, h4p, dis, consts, xsol, xlast)


def kernel(x, edge_index, edge_weights, Wo1, bo1, Wo2, bo2, Wol, bol,
           Wg1, bg1, Wg2, bg2, Wgl, bgl):
    row = edge_index[0].astype(jnp.int32)
    col = edge_index[1].astype(jnp.int32)
    ew = edge_weights.astype(jnp.float32)

    wg1a = Wg1[:, :D]
    wg1b = Wg1[:, D].reshape(1, D)
    rc = jnp.bitwise_or(jnp.left_shift(row, 16), col)

    deg_parts = _sc_deg(col, ew).reshape(NC, NPAD)
    dis1 = _tc_dis(deg_parts)[:N]
    dis = dis1.reshape(N, 1)

    h1p, hga = _tc_mm2(x, Wo1, wg1a, dis)

    s1 = _sc_agg(h1p, rc, ew).reshape(NC, NPAD, D)
    h2p = _tc_post_mm(s1, h1p, dis, bo1.reshape(1, D), Wo2, D)

    s2 = _sc_agg(h2p, rc, ew).reshape(NC, NPAD, D)
    xsol, hgp = _tc_xsol(s2, h2p, dis, bo2.reshape(1, D), Wol,
                         bol.reshape(1, D), hga, wg1b)

    s3 = _sc_agg(hgp, rc, ew).reshape(NC, NPAD, D)
    h4p = _tc_post_mm(s3, hgp, dis, bg1.reshape(1, D), Wg2, 1)

    s4 = _sc_agg1(h4p.reshape(N), row, col, ew).reshape(NC, NPAD)[:, :N]

    consts = jnp.stack([bg2[0], Wgl[0, 0], bgl[0]]).reshape(1, 3)
    xnew, gamma = _tc_final(s4.reshape(NC, N, 1), h4p, dis, consts,
                            xsol, x[:, D - 1:D])
    return (xnew, gamma)


# speedup vs baseline: 1.3682x; 1.0579x over previous
import functools

import jax
import jax.numpy as jnp
from jax import lax
from jax.experimental import pallas as pl
from jax.experimental.pallas import tpu as pltpu
from jax.experimental.pallas import tpu_sc as plsc

N = 10000
E = 320000
D = 128
NC, NS, L = 2, 16, 16
NW = NC * NS
EPT = E // NW
RPT = 632
RPT_LAST = N - 15 * RPT
NPAD = RPT * NS
K = 64
NCH = EPT // K
TAIL = EPT - NCH * K
NBUF = 3
_LOOPED = (NCH - 2) - (NCH - 2) % NBUF

_MESH = plsc.VectorSubcoreMesh(core_axis_name="c", subcore_axis_name="s")
_PREC = lax.Precision.HIGHEST


def _worker(base_len):
    cid = lax.axis_index("c")
    sid = lax.axis_index("s")
    wid = sid * NC + cid
    return cid, sid, pl.multiple_of(wid * base_len, 8)


def _zero_rows(z_v, acc_sh, start, nrows):
    nz = z_v.shape[0]
    off = 0
    while off < nrows:
        step = min(nz, nrows - off)
        pltpu.sync_copy(z_v.at[pl.ds(0, step)], acc_sh.at[pl.ds(start + off, step)])
        off += step


def _fill_zero(z_v):
    nz = z_v.shape[0]
    zero = jnp.zeros((L,), jnp.float32)

    if len(z_v.shape) == 1:
        @pl.loop(0, nz, step=L)
        def _(i):
            z_v[pl.ds(i, L)] = zero
    else:
        @pl.loop(0, nz)
        def _(j):
            for q in range(z_v.shape[1] // L):
                z_v[j, pl.ds(q * L, L)] = zero


def _drain_rows(acc_sh, out_hbm, stage, start, dst, nrows):
    off = 0
    nz = stage.shape[0]
    while off < nrows:
        step = min(nz, nrows - off)
        pltpu.sync_copy(acc_sh.at[pl.ds(start + off, step)], stage.at[pl.ds(0, step)])
        pltpu.sync_copy(stage.at[pl.ds(0, step)], out_hbm.at[pl.ds(dst + off, step)])
        off += step


def _per_tile_rows(sid, fn):
    start = pl.multiple_of(sid * RPT, 8)

    @pl.when(sid < NS - 1)
    def _():
        fn(start, RPT)

    @pl.when(sid == NS - 1)
    def _():
        fn(start, RPT_LAST)


@functools.partial(
    pl.kernel,
    out_type=jax.ShapeDtypeStruct((NC * NPAD,), jnp.float32),
    mesh=_MESH,
    scratch_types=[
        pltpu.VMEM_SHARED((N,), jnp.float32),
        pltpu.VMEM((EPT,), jnp.float32),
        pltpu.VMEM((EPT,), jnp.int32),
        pltpu.VMEM((RPT,), jnp.float32),
    ],
)
def _sc_deg(c_hbm, w_hbm, out_hbm, acc_sh, w_v, c_v, z_v):
    cid, sid, base = _worker(EPT)
    _fill_zero(z_v)
    _per_tile_rows(sid, lambda s, n: _zero_rows(z_v, acc_sh, s, n))
    plsc.subcore_barrier()
    pltpu.sync_copy(w_hbm.at[pl.ds(base, EPT)], w_v)
    pltpu.sync_copy(c_hbm.at[pl.ds(base, EPT)], c_v)
    pltpu.sync_copy(w_v, acc_sh.at[c_v], add=True)
    plsc.subcore_barrier()
    d0 = pl.multiple_of(cid * NPAD + sid * RPT, 8)
    _per_tile_rows(sid, lambda s, n: _drain_rows(acc_sh, out_hbm, z_v, s, d0, n))


@functools.partial(
    pl.kernel,
    out_type=jax.ShapeDtypeStruct((NC * NPAD, D), jnp.float32),
    mesh=_MESH,
    scratch_types=[
        pltpu.VMEM_SHARED((N, D), jnp.float32),
        pltpu.VMEM((K, D), jnp.float32),
        pltpu.VMEM((K, D), jnp.float32),
        pltpu.VMEM((K, D), jnp.float32),
        pltpu.VMEM((TAIL, D), jnp.float32),
        pltpu.VMEM((EPT,), jnp.int32),
        pltpu.VMEM((EPT,), jnp.float32),
        pltpu.VMEM((K,), jnp.int32),
        pltpu.VMEM((K,), jnp.int32),
        pltpu.VMEM((K,), jnp.int32),
        pltpu.VMEM((TAIL,), jnp.int32),
        pltpu.SemaphoreType.DMA,
        pltpu.SemaphoreType.DMA,
        pltpu.SemaphoreType.DMA,
        pltpu.SemaphoreType.DMA,
        pltpu.SemaphoreType.DMA,
        pltpu.SemaphoreType.DMA,
    ],
)
def _sc_agg(h_hbm, rc_hbm, w_hbm, out_hbm, acc_sh,
            rows0, rows1, rows2, rows_t, rc_all, w_all, i0, i1, i2, i_t,
            sg0, sg1, sg2, ss0, ss1, ss2):
    cid, sid, base = _worker(EPT)
    _fill_zero(rows0)
    _per_tile_rows(sid, lambda s, n: _zero_rows(rows0, acc_sh, s, n))
    pltpu.sync_copy(rc_hbm.at[pl.ds(base, EPT)], rc_all)
    pltpu.sync_copy(w_hbm.at[pl.ds(base, EPT)], w_all)
    plsc.subcore_barrier()

    bufs = ((rows0, i0, sg0, ss0), (rows1, i1, sg1, ss1), (rows2, i2, sg2, ss2))

    def start(k, buf):
        rows, i_v, sg, ss = buf
        off = pl.multiple_of(k * K, 8)
        for j in range(0, K, L):
            rc = rc_all[pl.ds(off + j, L)]
            i_v[pl.ds(j, L)] = lax.shift_right_logical(rc, 16)
        pltpu.async_copy(h_hbm.at[i_v], rows, sg)

    def wait_scatter(buf):
        rows, i_v, sg, ss = buf
        pltpu.make_async_copy(rows, acc_sh.at[i_v], ss).wait()

    def body(k, buf):
        rows, i_v, sg, ss = buf
        pltpu.make_async_copy(h_hbm.at[i_v], rows, sg).wait()
        off = pl.multiple_of(k * K, 8)
        for j in range(0, K, L):
            rc = rc_all[pl.ds(off + j, L)]
            i_v[pl.ds(j, L)] = lax.bitwise_and(rc, jnp.int32(0xFFFF))

        @pl.loop(0, K, step=L)
        def _(j0):
            wv = w_all[pl.ds(off + j0, L)]
            for t in range(L):
                s = wv[t]
                for q in range(D // L):
                    rows[j0 + t, pl.ds(q * L, L)] = rows[j0 + t, pl.ds(q * L, L)] * s

        pltpu.async_copy(rows, acc_sh.at[i_v], ss, add=True)

    start(0, bufs[0])

    @pl.loop(0, _LOOPED, step=NBUF)
    def _(kk):
        for b in range(NBUF):
            k = kk + b
            bn = (b + 1) % NBUF

            @pl.when(k >= 2)
            def _():
                wait_scatter(bufs[bn])

            start(k + 1, bufs[bn])
            body(k, bufs[b])

    for k in range(_LOOPED, NCH):
        bn = (k + 1) % NBUF
        if k >= 2:
            wait_scatter(bufs[bn])
        if k + 1 < NCH:
            start(k + 1, bufs[bn])
        body(k, bufs[k % NBUF])
    wait_scatter(bufs[(NCH - 2) % NBUF])
    wait_scatter(bufs[(NCH - 1) % NBUF])

    toff = NCH * K
    rc = rc_all[pl.ds(toff, L)]
    i_t[pl.ds(0, L)] = lax.shift_right_logical(rc, 16)
    pltpu.async_copy(h_hbm.at[i_t], rows_t, sg0).wait()
    i_t[pl.ds(0, L)] = lax.bitwise_and(rc, jnp.int32(0xFFFF))
    wv = w_all[pl.ds(toff, L)]
    for t in range(L):
        s = wv[t]
        for q in range(D // L):
            rows_t[t, pl.ds(q * L, L)] = rows_t[t, pl.ds(q * L, L)] * s
    pltpu.sync_copy(rows_t, acc_sh.at[i_t], add=True)

    plsc.subcore_barrier()
    d0 = pl.multiple_of(cid * NPAD + sid * RPT, 8)
    _per_tile_rows(sid, lambda s, n: _drain_rows(acc_sh, out_hbm, rows0, s, d0, n))


@functools.partial(
    pl.kernel,
    out_type=jax.ShapeDtypeStruct((NC * NPAD,), jnp.float32),
    mesh=_MESH,
    compiler_params=pltpu.CompilerParams(needs_layout_passes=False),
    scratch_types=[
        pltpu.VMEM_SHARED((N,), jnp.float32),
        pltpu.VMEM((EPT,), jnp.int32),
        pltpu.VMEM((EPT,), jnp.int32),
        pltpu.VMEM((EPT,), jnp.float32),
        pltpu.VMEM((EPT,), jnp.float32),
        pltpu.VMEM((N,), jnp.float32),
        pltpu.VMEM((RPT,), jnp.float32),
        pltpu.SemaphoreType.DMA,
    ],
)
def _sc_agg1(g_hbm, r_hbm, c_hbm, w_hbm, out_hbm, acc_sh,
             r_v, c_v, w_v, g_v, h4_all, z_v, sem):
    cid, sid, base = _worker(EPT)
    _fill_zero(z_v)
    _per_tile_rows(sid, lambda s, n: _zero_rows(z_v, acc_sh, s, n))
    plsc.subcore_barrier()
    pltpu.sync_copy(r_hbm.at[pl.ds(base, EPT)], r_v)
    pltpu.sync_copy(c_hbm.at[pl.ds(base, EPT)], c_v)
    pltpu.sync_copy(w_hbm.at[pl.ds(base, EPT)], w_v)
    pltpu.sync_copy(g_hbm, h4_all)

    @pl.loop(0, EPT, step=L)
    def _(i):
        idx = r_v[pl.ds(i, L)]
        vals = plsc.load_gather(h4_all, [idx])
        g_v[pl.ds(i, L)] = vals * w_v[pl.ds(i, L)]

    pltpu.sync_copy(g_v, acc_sh.at[c_v], add=True)
    plsc.subcore_barrier()
    d0 = pl.multiple_of(cid * NPAD + sid * RPT, 8)
    _per_tile_rows(sid, lambda s, n: _drain_rows(acc_sh, out_hbm, z_v, s, d0, n))


_BLK = 1000
_GRID = N // _BLK


def _dis_body(deg_ref, dis_ref):
    d = deg_ref[0, :] + deg_ref[1, :] + 1.0
    dis_ref[...] = lax.rsqrt(d)


def _tc_dis(deg_parts):
    return pl.pallas_call(
        _dis_body,
        out_shape=jax.ShapeDtypeStruct((NPAD,), jnp.float32),
    )(deg_parts)


def _mm2_body(x_ref, wa_ref, wb_ref, dis_ref, hp_ref, hb_ref):
    x = x_ref[...]
    d = dis_ref[...]
    ha = lax.dot_general(x, wa_ref[...], (((1,), (1,)), ((), ())),
                         precision=_PREC, preferred_element_type=jnp.float32)
    hp_ref[...] = d * ha
    hb_ref[...] = lax.dot_general(x, wb_ref[...], (((1,), (1,)), ((), ())),
                                  precision=_PREC,
                                  preferred_element_type=jnp.float32)


def _tc_mm2(x, wa, wb, dis):
    return pl.pallas_call(
        _mm2_body,
        grid=(_GRID,),
        in_specs=[
            pl.BlockSpec((_BLK, D), lambda i: (i, 0)),
            pl.BlockSpec((D, D), lambda i: (0, 0)),
            pl.BlockSpec((D, D), lambda i: (0, 0)),
            pl.BlockSpec((_BLK, 1), lambda i: (i, 0)),
        ],
        out_specs=[
            pl.BlockSpec((_BLK, D), lambda i: (i, 0)),
            pl.BlockSpec((_BLK, D), lambda i: (i, 0)),
        ],
        out_shape=[
            jax.ShapeDtypeStruct((N, D), jnp.float32),
            jax.ShapeDtypeStruct((N, D), jnp.float32),
        ],
    )(x, wa, wb, dis)


def _post_mm_body(s_ref, h_ref, dis_ref, b_ref, w_ref, out_ref):
    d = dis_ref[...]
    z = jnp.maximum(d * (s_ref[0] + s_ref[1] + h_ref[...]) + b_ref[...], 0.0)
    o = lax.dot_general(z, w_ref[...], (((1,), (1,)), ((), ())),
                        precision=_PREC, preferred_element_type=jnp.float32)
    out_ref[...] = d * o


def _tc_post_mm(s_parts, hp, dis, b, w, d_out):
    return pl.pallas_call(
        _post_mm_body,
        grid=(_GRID,),
        in_specs=[
            pl.BlockSpec((2, _BLK, D), lambda i: (0, i, 0)),
            pl.BlockSpec((_BLK, D), lambda i: (i, 0)),
            pl.BlockSpec((_BLK, 1), lambda i: (i, 0)),
            pl.BlockSpec((1, D), lambda i: (0, 0)),
            pl.BlockSpec((d_out, D), lambda i: (0, 0)),
        ],
        out_specs=pl.BlockSpec((_BLK, d_out), lambda i: (i, 0)),
        out_shape=jax.ShapeDtypeStruct((N, d_out), jnp.float32),
    )(s_parts, hp, dis, b, w)


def _xsol_body(s_ref, h_ref, dis_ref, b_ref, wol_ref, bol_ref, hga_ref,
               wg1b_ref, xsol_ref, hgp_ref):
    d = dis_ref[...]
    z = d * (s_ref[0] + s_ref[1] + h_ref[...]) + b_ref[...]
    xsol = lax.dot_general(z, wol_ref[...], (((1,), (1,)), ((), ())),
                           precision=_PREC,
                           preferred_element_type=jnp.float32) + bol_ref[...]
    xsol_ref[...] = xsol
    hgp_ref[...] = d * (hga_ref[...] + xsol[:, D - 1:D] * wg1b_ref[...])


def _tc_xsol(s_parts, hp, dis, b, wol, bol, hga, wg1b):
    return pl.pallas_call(
        _xsol_body,
        grid=(_GRID,),
        in_specs=[
            pl.BlockSpec((2, _BLK, D), lambda i: (0, i, 0)),
            pl.BlockSpec((_BLK, D), lambda i: (i, 0)),
            pl.BlockSpec((_BLK, 1), lambda i: (i, 0)),
            pl.BlockSpec((1, D), lambda i: (0, 0)),
            pl.BlockSpec((D, D), lambda i: (0, 0)),
            pl.BlockSpec((1, D), lambda i: (0, 0)),
            pl.BlockSpec((_BLK, D), lambda i: (i, 0)),
            pl.BlockSpec((1, D), lambda i: (0, 0)),
        ],
        out_specs=[
            pl.BlockSpec((_BLK, D), lambda i: (i, 0)),
            pl.BlockSpec((_BLK, D), lambda i: (i, 0)),
        ],
        out_shape=[
            jax.ShapeDtypeStruct((N, D), jnp.float32),
            jax.ShapeDtypeStruct((N, D), jnp.float32),
        ],
    )(s_parts, hp, dis, b, wol, bol, hga, wg1b)


def _final_body(s_ref, h4p_ref, dis_ref, consts_ref, xsol_ref, xlast_ref,
                xnew_ref, gamma_ref):
    d = dis_ref[...]
    bg2 = consts_ref[0, 0]
    wgl = consts_ref[0, 1]
    bgl = consts_ref[0, 2]
    g = d * (s_ref[0] + s_ref[1] + h4p_ref[...]) + bg2
    gl = g * wgl + bgl
    gmin = jnp.min(gl)
    gmax = jnp.max(gl)
    gamma = (gl - gmin) / (gmax - gmin + 1e-12)
    gamma_ref[...] = gamma
    xsol = xsol_ref[...]
    xl = xlast_ref[...]
    upd = xl + gamma * (xsol[:, D - 1:D] - xl)
    col = lax.broadcasted_iota(jnp.int32, (N, D), 1)
    xnew_ref[...] = jnp.where(col == D - 1, upd, xsol)


def _tc_final(s_parts, h4p, dis, consts, xsol, xlast):
    return pl.pallas_call(
        _final_body,
        out_shape=[
            jax.ShapeDtypeStruct((N, D), jnp.float32),
            jax.ShapeDtypeStruct((N, 1), jnp.float32),
        ],
    )(s_parts, h4p, dis, consts, xsol, xlast)


def kernel(x, edge_index, edge_weights, Wo1, bo1, Wo2, bo2, Wol, bol,
           Wg1, bg1, Wg2, bg2, Wgl, bgl):
    row = edge_index[0].astype(jnp.int32)
    col = edge_index[1].astype(jnp.int32)
    ew = edge_weights.astype(jnp.float32)

    wg1a = Wg1[:, :D]
    wg1b = Wg1[:, D].reshape(1, D)
    rc = jnp.bitwise_or(jnp.left_shift(row, 16), col)

    deg_parts = _sc_deg(col, ew).reshape(NC, NPAD)
    dis1 = _tc_dis(deg_parts)[:N]
    dis = dis1.reshape(N, 1)

    h1p, hga = _tc_mm2(x, Wo1, wg1a, dis)

    s1 = _sc_agg(h1p, rc, ew).reshape(NC, NPAD, D)
    h2p = _tc_post_mm(s1, h1p, dis, bo1.reshape(1, D), Wo2, D)

    s2 = _sc_agg(h2p, rc, ew).reshape(NC, NPAD, D)
    xsol, hgp = _tc_xsol(s2, h2p, dis, bo2.reshape(1, D), Wol,
                         bol.reshape(1, D), hga, wg1b)

    s3 = _sc_agg(hgp, rc, ew).reshape(NC, NPAD, D)
    h4p = _tc_post_mm(s3, hgp, dis, bg1.reshape(1, D), Wg2, 1)

    s4 = _sc_agg1(h4p.reshape(N), row, col, ew).reshape(NC, NPAD)[:, :N]

    consts = jnp.stack([bg2[0], Wgl[0, 0], bgl[0]]).reshape(1, 3)
    xnew, gamma = _tc_final(s4.reshape(NC, N, 1), h4p, dis, consts,
                            xsol, x[:, D - 1:D])
    return (xnew, gamma)


# speedup vs baseline: 1.4799x; 1.0816x over previous
import functools

import jax
import jax.numpy as jnp
from jax import lax
from jax.experimental import pallas as pl
from jax.experimental.pallas import tpu as pltpu
from jax.experimental.pallas import tpu_sc as plsc

N = 10000
E = 320000
D = 128
NC, NS, L = 2, 16, 16
NW = NC * NS
EPT = E // NW
RPT = 632
RPT_LAST = N - 15 * RPT
NPAD = RPT * NS
K = 48
NCH = EPT // K
TAIL = EPT - NCH * K
NBUF = 4
_LOOPED = (NCH - 2) - (NCH - 2) % NBUF

_MESH = plsc.VectorSubcoreMesh(core_axis_name="c", subcore_axis_name="s")
_PREC = lax.Precision.HIGHEST


def _worker(base_len):
    cid = lax.axis_index("c")
    sid = lax.axis_index("s")
    wid = sid * NC + cid
    return cid, sid, pl.multiple_of(wid * base_len, 8)


def _zero_rows(z_v, acc_sh, start, nrows):
    nz = z_v.shape[0]
    off = 0
    while off < nrows:
        step = min(nz, nrows - off)
        pltpu.sync_copy(z_v.at[pl.ds(0, step)], acc_sh.at[pl.ds(start + off, step)])
        off += step


def _fill_zero(z_v):
    nz = z_v.shape[0]
    zero = jnp.zeros((L,), jnp.float32)

    if len(z_v.shape) == 1:
        @pl.loop(0, nz, step=L)
        def _(i):
            z_v[pl.ds(i, L)] = zero
    else:
        @pl.loop(0, nz)
        def _(j):
            for q in range(z_v.shape[1] // L):
                z_v[j, pl.ds(q * L, L)] = zero


def _drain_rows(acc_sh, out_hbm, stage, start, dst, nrows):
    off = 0
    nz = stage.shape[0]
    while off < nrows:
        step = min(nz, nrows - off)
        pltpu.sync_copy(acc_sh.at[pl.ds(start + off, step)], stage.at[pl.ds(0, step)])
        pltpu.sync_copy(stage.at[pl.ds(0, step)], out_hbm.at[pl.ds(dst + off, step)])
        off += step


def _per_tile_rows(sid, fn):
    start = pl.multiple_of(sid * RPT, 8)

    @pl.when(sid < NS - 1)
    def _():
        fn(start, RPT)

    @pl.when(sid == NS - 1)
    def _():
        fn(start, RPT_LAST)


@functools.partial(
    pl.kernel,
    out_type=jax.ShapeDtypeStruct((NC * NPAD,), jnp.float32),
    mesh=_MESH,
    scratch_types=[
        pltpu.VMEM_SHARED((N,), jnp.float32),
        pltpu.VMEM((EPT,), jnp.float32),
        pltpu.VMEM((EPT,), jnp.int32),
        pltpu.VMEM((RPT,), jnp.float32),
    ],
)
def _sc_deg(c_hbm, w_hbm, out_hbm, acc_sh, w_v, c_v, z_v):
    cid, sid, base = _worker(EPT)
    _fill_zero(z_v)
    _per_tile_rows(sid, lambda s, n: _zero_rows(z_v, acc_sh, s, n))
    plsc.subcore_barrier()
    pltpu.sync_copy(w_hbm.at[pl.ds(base, EPT)], w_v)
    pltpu.sync_copy(c_hbm.at[pl.ds(base, EPT)], c_v)
    pltpu.sync_copy(w_v, acc_sh.at[c_v], add=True)
    plsc.subcore_barrier()
    d0 = pl.multiple_of(cid * NPAD + sid * RPT, 8)
    _per_tile_rows(sid, lambda s, n: _drain_rows(acc_sh, out_hbm, z_v, s, d0, n))


@functools.partial(
    pl.kernel,
    out_type=jax.ShapeDtypeStruct((NC * NPAD, D), jnp.float32),
    mesh=_MESH,
    scratch_types=[
        pltpu.VMEM_SHARED((N, D), jnp.float32),
        pltpu.VMEM((K, D), jnp.float32),
        pltpu.VMEM((K, D), jnp.float32),
        pltpu.VMEM((K, D), jnp.float32),
        pltpu.VMEM((K, D), jnp.float32),
        pltpu.VMEM((TAIL, D), jnp.float32),
        pltpu.VMEM((EPT,), jnp.int32),
        pltpu.VMEM((EPT,), jnp.float32),
        pltpu.VMEM((K,), jnp.int32),
        pltpu.VMEM((K,), jnp.int32),
        pltpu.VMEM((K,), jnp.int32),
        pltpu.VMEM((K,), jnp.int32),
        pltpu.VMEM((TAIL,), jnp.int32),
        pltpu.SemaphoreType.DMA,
        pltpu.SemaphoreType.DMA,
        pltpu.SemaphoreType.DMA,
        pltpu.SemaphoreType.DMA,
        pltpu.SemaphoreType.DMA,
        pltpu.SemaphoreType.DMA,
        pltpu.SemaphoreType.DMA,
        pltpu.SemaphoreType.DMA,
    ],
)
def _sc_agg(h_hbm, rc_hbm, w_hbm, out_hbm, acc_sh,
            rows0, rows1, rows2, rows3, rows_t, rc_all, w_all,
            i0, i1, i2, i3, i_t,
            sg0, sg1, sg2, sg3, ss0, ss1, ss2, ss3):
    cid, sid, base = _worker(EPT)
    _fill_zero(rows0)
    _per_tile_rows(sid, lambda s, n: _zero_rows(rows0, acc_sh, s, n))
    pltpu.sync_copy(rc_hbm.at[pl.ds(base, EPT)], rc_all)
    pltpu.sync_copy(w_hbm.at[pl.ds(base, EPT)], w_all)
    plsc.subcore_barrier()

    bufs = ((rows0, i0, sg0, ss0), (rows1, i1, sg1, ss1),
            (rows2, i2, sg2, ss2), (rows3, i3, sg3, ss3))

    def start(k, buf):
        rows, i_v, sg, ss = buf
        off = pl.multiple_of(k * K, 8)
        for j in range(0, K, L):
            rc = rc_all[pl.ds(off + j, L)]
            i_v[pl.ds(j, L)] = lax.shift_right_logical(rc, 16)
        pltpu.async_copy(h_hbm.at[i_v], rows, sg)

    def wait_scatter(buf):
        rows, i_v, sg, ss = buf
        pltpu.make_async_copy(rows, acc_sh.at[i_v], ss).wait()

    def body(k, buf):
        rows, i_v, sg, ss = buf
        pltpu.make_async_copy(h_hbm.at[i_v], rows, sg).wait()
        off = pl.multiple_of(k * K, 8)
        for j in range(0, K, L):
            rc = rc_all[pl.ds(off + j, L)]
            i_v[pl.ds(j, L)] = lax.bitwise_and(rc, jnp.int32(0xFFFF))

        @pl.loop(0, K, step=L)
        def _(j0):
            wv = w_all[pl.ds(off + j0, L)]
            for t in range(L):
                s = wv[t]
                for q in range(D // L):
                    rows[j0 + t, pl.ds(q * L, L)] = rows[j0 + t, pl.ds(q * L, L)] * s

        pltpu.async_copy(rows, acc_sh.at[i_v], ss, add=True)

    start(0, bufs[0])
    start(1, bufs[1])

    @pl.loop(0, _LOOPED, step=NBUF)
    def _(kk):
        for b in range(NBUF):
            k = kk + b
            bn = (b + 2) % NBUF

            @pl.when(k >= 2)
            def _():
                wait_scatter(bufs[bn])

            start(k + 2, bufs[bn])
            body(k, bufs[b])

    for k in range(_LOOPED, NCH):
        bn = (k + 2) % NBUF
        if k >= 2:
            wait_scatter(bufs[bn])
        if k + 2 < NCH:
            start(k + 2, bufs[bn])
        body(k, bufs[k % NBUF])
    wait_scatter(bufs[(NCH - 2) % NBUF])
    wait_scatter(bufs[(NCH - 1) % NBUF])

    toff = NCH * K
    rc = rc_all[pl.ds(toff, L)]
    i_t[pl.ds(0, L)] = lax.shift_right_logical(rc, 16)
    pltpu.async_copy(h_hbm.at[i_t], rows_t, sg0).wait()
    i_t[pl.ds(0, L)] = lax.bitwise_and(rc, jnp.int32(0xFFFF))
    wv = w_all[pl.ds(toff, L)]
    for t in range(L):
        s = wv[t]
        for q in range(D // L):
            rows_t[t, pl.ds(q * L, L)] = rows_t[t, pl.ds(q * L, L)] * s
    pltpu.sync_copy(rows_t, acc_sh.at[i_t], add=True)

    plsc.subcore_barrier()
    d0 = pl.multiple_of(cid * NPAD + sid * RPT, 8)
    _per_tile_rows(sid, lambda s, n: _drain_rows(acc_sh, out_hbm, rows0, s, d0, n))


@functools.partial(
    pl.kernel,
    out_type=jax.ShapeDtypeStruct((NC * NPAD,), jnp.float32),
    mesh=_MESH,
    compiler_params=pltpu.CompilerParams(needs_layout_passes=False),
    scratch_types=[
        pltpu.VMEM_SHARED((N,), jnp.float32),
        pltpu.VMEM((EPT,), jnp.int32),
        pltpu.VMEM((EPT,), jnp.int32),
        pltpu.VMEM((EPT,), jnp.float32),
        pltpu.VMEM((EPT,), jnp.float32),
        pltpu.VMEM((N,), jnp.float32),
        pltpu.VMEM((RPT,), jnp.float32),
        pltpu.SemaphoreType.DMA,
    ],
)
def _sc_agg1(g_hbm, r_hbm, c_hbm, w_hbm, out_hbm, acc_sh,
             r_v, c_v, w_v, g_v, h4_all, z_v, sem):
    cid, sid, base = _worker(EPT)
    _fill_zero(z_v)
    _per_tile_rows(sid, lambda s, n: _zero_rows(z_v, acc_sh, s, n))
    plsc.subcore_barrier()
    pltpu.sync_copy(r_hbm.at[pl.ds(base, EPT)], r_v)
    pltpu.sync_copy(c_hbm.at[pl.ds(base, EPT)], c_v)
    pltpu.sync_copy(w_hbm.at[pl.ds(base, EPT)], w_v)
    pltpu.sync_copy(g_hbm, h4_all)

    @pl.loop(0, EPT, step=L)
    def _(i):
        idx = r_v[pl.ds(i, L)]
        vals = plsc.load_gather(h4_all, [idx])
        g_v[pl.ds(i, L)] = vals * w_v[pl.ds(i, L)]

    pltpu.sync_copy(g_v, acc_sh.at[c_v], add=True)
    plsc.subcore_barrier()
    d0 = pl.multiple_of(cid * NPAD + sid * RPT, 8)
    _per_tile_rows(sid, lambda s, n: _drain_rows(acc_sh, out_hbm, z_v, s, d0, n))


_BLK = 1000
_GRID = N // _BLK


def _dis_body(deg_ref, dis_ref):
    d = deg_ref[0, :] + deg_ref[1, :] + 1.0
    dis_ref[...] = lax.rsqrt(d)


def _tc_dis(deg_parts):
    return pl.pallas_call(
        _dis_body,
        out_shape=jax.ShapeDtypeStruct((NPAD,), jnp.float32),
    )(deg_parts)


def _mm2_body(x_ref, wa_ref, wb_ref, dis_ref, hp_ref, hb_ref):
    x = x_ref[...]
    d = dis_ref[...]
    ha = lax.dot_general(x, wa_ref[...], (((1,), (1,)), ((), ())),
                         precision=_PREC, preferred_element_type=jnp.float32)
    hp_ref[...] = d * ha
    hb_ref[...] = lax.dot_general(x, wb_ref[...], (((1,), (1,)), ((), ())),
                                  precision=_PREC,
                                  preferred_element_type=jnp.float32)


def _tc_mm2(x, wa, wb, dis):
    return pl.pallas_call(
        _mm2_body,
        grid=(_GRID,),
        in_specs=[
            pl.BlockSpec((_BLK, D), lambda i: (i, 0)),
            pl.BlockSpec((D, D), lambda i: (0, 0)),
            pl.BlockSpec((D, D), lambda i: (0, 0)),
            pl.BlockSpec((_BLK, 1), lambda i: (i, 0)),
        ],
        out_specs=[
            pl.BlockSpec((_BLK, D), lambda i: (i, 0)),
            pl.BlockSpec((_BLK, D), lambda i: (i, 0)),
        ],
        out_shape=[
            jax.ShapeDtypeStruct((N, D), jnp.float32),
            jax.ShapeDtypeStruct((N, D), jnp.float32),
        ],
    )(x, wa, wb, dis)


def _post_mm_body(s_ref, h_ref, dis_ref, b_ref, w_ref, out_ref):
    d = dis_ref[...]
    z = jnp.maximum(d * (s_ref[0] + s_ref[1] + h_ref[...]) + b_ref[...], 0.0)
    o = lax.dot_general(z, w_ref[...], (((1,), (1,)), ((), ())),
                        precision=_PREC, preferred_element_type=jnp.float32)
    out_ref[...] = d * o


def _tc_post_mm(s_parts, hp, dis, b, w, d_out):
    return pl.pallas_call(
        _post_mm_body,
        grid=(_GRID,),
        in_specs=[
            pl.BlockSpec((2, _BLK, D), lambda i: (0, i, 0)),
            pl.BlockSpec((_BLK, D), lambda i: (i, 0)),
            pl.BlockSpec((_BLK, 1), lambda i: (i, 0)),
            pl.BlockSpec((1, D), lambda i: (0, 0)),
            pl.BlockSpec((d_out, D), lambda i: (0, 0)),
        ],
        out_specs=pl.BlockSpec((_BLK, d_out), lambda i: (i, 0)),
        out_shape=jax.ShapeDtypeStruct((N, d_out), jnp.float32),
    )(s_parts, hp, dis, b, w)


def _xsol_body(s_ref, h_ref, dis_ref, b_ref, wol_ref, bol_ref, hga_ref,
               wg1b_ref, xsol_ref, hgp_ref):
    d = dis_ref[...]
    z = d * (s_ref[0] + s_ref[1] + h_ref[...]) + b_ref[...]
    xsol = lax.dot_general(z, wol_ref[...], (((1,), (1,)), ((), ())),
                           precision=_PREC,
                           preferred_element_type=jnp.float32) + bol_ref[...]
    xsol_ref[...] = xsol
    hgp_ref[...] = d * (hga_ref[...] + xsol[:, D - 1:D] * wg1b_ref[...])


def _tc_xsol(s_parts, hp, dis, b, wol, bol, hga, wg1b):
    return pl.pallas_call(
        _xsol_body,
        grid=(_GRID,),
        in_specs=[
            pl.BlockSpec((2, _BLK, D), lambda i: (0, i, 0)),
            pl.BlockSpec((_BLK, D), lambda i: (i, 0)),
            pl.BlockSpec((_BLK, 1), lambda i: (i, 0)),
            pl.BlockSpec((1, D), lambda i: (0, 0)),
            pl.BlockSpec((D, D), lambda i: (0, 0)),
            pl.BlockSpec((1, D), lambda i: (0, 0)),
            pl.BlockSpec((_BLK, D), lambda i: (i, 0)),
            pl.BlockSpec((1, D), lambda i: (0, 0)),
        ],
        out_specs=[
            pl.BlockSpec((_BLK, D), lambda i: (i, 0)),
            pl.BlockSpec((_BLK, D), lambda i: (i, 0)),
        ],
        out_shape=[
            jax.ShapeDtypeStruct((N, D), jnp.float32),
            jax.ShapeDtypeStruct((N, D), jnp.float32),
        ],
    )(s_parts, hp, dis, b, wol, bol, hga, wg1b)


def _final_body(s_ref, h4p_ref, dis_ref, consts_ref, xsol_ref, xlast_ref,
                xnew_ref, gamma_ref):
    d = dis_ref[...]
    bg2 = consts_ref[0, 0]
    wgl = consts_ref[0, 1]
    bgl = consts_ref[0, 2]
    g = d * (s_ref[0] + s_ref[1] + h4p_ref[...]) + bg2
    gl = g * wgl + bgl
    gmin = jnp.min(gl)
    gmax = jnp.max(gl)
    gamma = (gl - gmin) / (gmax - gmin + 1e-12)
    gamma_ref[...] = gamma
    xsol = xsol_ref[...]
    xl = xlast_ref[...]
    upd = xl + gamma * (xsol[:, D - 1:D] - xl)
    col = lax.broadcasted_iota(jnp.int32, (N, D), 1)
    xnew_ref[...] = jnp.where(col == D - 1, upd, xsol)


def _tc_final(s_parts, h4p, dis, consts, xsol, xlast):
    return pl.pallas_call(
        _final_body,
        out_shape=[
            jax.ShapeDtypeStruct((N, D), jnp.float32),
            jax.ShapeDtypeStruct((N, 1), jnp.float32),
        ],
    )(s_parts, h4p, dis, consts, xsol, xlast)


def kernel(x, edge_index, edge_weights, Wo1, bo1, Wo2, bo2, Wol, bol,
           Wg1, bg1, Wg2, bg2, Wgl, bgl):
    row = edge_index[0].astype(jnp.int32)
    col = edge_index[1].astype(jnp.int32)
    ew = edge_weights.astype(jnp.float32)

    wg1a = Wg1[:, :D]
    wg1b = Wg1[:, D].reshape(1, D)
    rc = jnp.bitwise_or(jnp.left_shift(row, 16), col)

    deg_parts = _sc_deg(col, ew).reshape(NC, NPAD)
    dis1 = _tc_dis(deg_parts)[:N]
    dis = dis1.reshape(N, 1)

    h1p, hga = _tc_mm2(x, Wo1, wg1a, dis)

    s1 = _sc_agg(h1p, rc, ew).reshape(NC, NPAD, D)
    h2p = _tc_post_mm(s1, h1p, dis, bo1.reshape(1, D), Wo2, D)

    s2 = _sc_agg(h2p, rc, ew).reshape(NC, NPAD, D)
    xsol, hgp = _tc_xsol(s2, h2p, dis, bo2.reshape(1, D), Wol,
                         bol.reshape(1, D), hga, wg1b)

    s3 = _sc_agg(hgp, rc, ew).reshape(NC, NPAD, D)
    h4p = _tc_post_mm(s3, hgp, dis, bg1.reshape(1, D), Wg2, 1)

    s4 = _sc_agg1(h4p.reshape(N), row, col, ew).reshape(NC, NPAD)[:, :N]

    consts = jnp.stack([bg2[0], Wgl[0, 0], bgl[0]]).reshape(1, 3)
    xnew, gamma = _tc_final(s4.reshape(NC, N, 1), h4p, dis, consts,
                            xsol, x[:, D - 1:D])
    return (xnew, gamma)


# speedup vs baseline: 1.4910x; 1.0075x over previous
import functools

import jax
import jax.numpy as jnp
from jax import lax
from jax.experimental import pallas as pl
from jax.experimental.pallas import tpu as pltpu
from jax.experimental.pallas import tpu_sc as plsc

N = 10000
E = 320000
D = 128
NC, NS, L = 2, 16, 16
NW = NC * NS
EPT = E // NW
RPT = 632
RPT_LAST = N - 15 * RPT
NPAD = RPT * NS
K = 32
NCH = EPT // K
TAIL = EPT - NCH * K
NBUF = 6
DEPTH = NBUF // 2
_LOOPED = (NCH - DEPTH) - (NCH - DEPTH) % NBUF

_MESH = plsc.VectorSubcoreMesh(core_axis_name="c", subcore_axis_name="s")
_PREC = lax.Precision.HIGHEST


def _worker(base_len):
    cid = lax.axis_index("c")
    sid = lax.axis_index("s")
    wid = sid * NC + cid
    return cid, sid, pl.multiple_of(wid * base_len, 8)


def _zero_rows(z_v, acc_sh, start, nrows):
    nz = z_v.shape[0]
    off = 0
    while off < nrows:
        step = min(nz, nrows - off)
        pltpu.sync_copy(z_v.at[pl.ds(0, step)], acc_sh.at[pl.ds(start + off, step)])
        off += step


def _fill_zero(z_v):
    nz = z_v.shape[0]
    zero = jnp.zeros((L,), jnp.float32)

    if len(z_v.shape) == 1:
        @pl.loop(0, nz, step=L)
        def _(i):
            z_v[pl.ds(i, L)] = zero
    else:
        @pl.loop(0, nz)
        def _(j):
            for q in range(z_v.shape[1] // L):
                z_v[j, pl.ds(q * L, L)] = zero


def _drain_rows(acc_sh, out_hbm, stage, start, dst, nrows):
    off = 0
    nz = stage.shape[0]
    while off < nrows:
        step = min(nz, nrows - off)
        pltpu.sync_copy(acc_sh.at[pl.ds(start + off, step)], stage.at[pl.ds(0, step)])
        pltpu.sync_copy(stage.at[pl.ds(0, step)], out_hbm.at[pl.ds(dst + off, step)])
        off += step


def _per_tile_rows(sid, fn):
    start = pl.multiple_of(sid * RPT, 8)

    @pl.when(sid < NS - 1)
    def _():
        fn(start, RPT)

    @pl.when(sid == NS - 1)
    def _():
        fn(start, RPT_LAST)


@functools.partial(
    pl.kernel,
    out_type=jax.ShapeDtypeStruct((NC * NPAD,), jnp.float32),
    mesh=_MESH,
    scratch_types=[
        pltpu.VMEM_SHARED((N,), jnp.float32),
        pltpu.VMEM((EPT,), jnp.float32),
        pltpu.VMEM((EPT,), jnp.int32),
        pltpu.VMEM((RPT,), jnp.float32),
    ],
)
def _sc_deg(c_hbm, w_hbm, out_hbm, acc_sh, w_v, c_v, z_v):
    cid, sid, base = _worker(EPT)
    _fill_zero(z_v)
    _per_tile_rows(sid, lambda s, n: _zero_rows(z_v, acc_sh, s, n))
    plsc.subcore_barrier()
    pltpu.sync_copy(w_hbm.at[pl.ds(base, EPT)], w_v)
    pltpu.sync_copy(c_hbm.at[pl.ds(base, EPT)], c_v)
    pltpu.sync_copy(w_v, acc_sh.at[c_v], add=True)
    plsc.subcore_barrier()
    d0 = pl.multiple_of(cid * NPAD + sid * RPT, 8)
    _per_tile_rows(sid, lambda s, n: _drain_rows(acc_sh, out_hbm, z_v, s, d0, n))


@functools.partial(
    pl.kernel,
    out_type=jax.ShapeDtypeStruct((NC * NPAD, D), jnp.float32),
    mesh=_MESH,
    scratch_types=[
        pltpu.VMEM_SHARED((N, D), jnp.float32),
        pltpu.VMEM((K, D), jnp.float32),
        pltpu.VMEM((K, D), jnp.float32),
        pltpu.VMEM((K, D), jnp.float32),
        pltpu.VMEM((K, D), jnp.float32),
        pltpu.VMEM((K, D), jnp.float32),
        pltpu.VMEM((K, D), jnp.float32),
        pltpu.VMEM((TAIL, D), jnp.float32),
        pltpu.VMEM((EPT,), jnp.int32),
        pltpu.VMEM((EPT,), jnp.float32),
        pltpu.VMEM((K,), jnp.int32),
        pltpu.VMEM((K,), jnp.int32),
        pltpu.VMEM((K,), jnp.int32),
        pltpu.VMEM((K,), jnp.int32),
        pltpu.VMEM((K,), jnp.int32),
        pltpu.VMEM((K,), jnp.int32),
        pltpu.VMEM((TAIL,), jnp.int32),
        pltpu.SemaphoreType.DMA,
        pltpu.SemaphoreType.DMA,
        pltpu.SemaphoreType.DMA,
        pltpu.SemaphoreType.DMA,
        pltpu.SemaphoreType.DMA,
        pltpu.SemaphoreType.DMA,
        pltpu.SemaphoreType.DMA,
        pltpu.SemaphoreType.DMA,
        pltpu.SemaphoreType.DMA,
        pltpu.SemaphoreType.DMA,
        pltpu.SemaphoreType.DMA,
        pltpu.SemaphoreType.DMA,
    ],
)
def _sc_agg(h_hbm, rc_hbm, w_hbm, out_hbm, acc_sh,
            rows0, rows1, rows2, rows3, rows4, rows5, rows_t, rc_all, w_all,
            i0, i1, i2, i3, i4, i5, i_t,
            sg0, sg1, sg2, sg3, sg4, sg5, ss0, ss1, ss2, ss3, ss4, ss5):
    cid, sid, base = _worker(EPT)
    _fill_zero(rows0)
    _per_tile_rows(sid, lambda s, n: _zero_rows(rows0, acc_sh, s, n))
    pltpu.sync_copy(rc_hbm.at[pl.ds(base, EPT)], rc_all)
    pltpu.sync_copy(w_hbm.at[pl.ds(base, EPT)], w_all)
    plsc.subcore_barrier()

    bufs = ((rows0, i0, sg0, ss0), (rows1, i1, sg1, ss1),
            (rows2, i2, sg2, ss2), (rows3, i3, sg3, ss3),
            (rows4, i4, sg4, ss4), (rows5, i5, sg5, ss5))

    def start(k, buf):
        rows, i_v, sg, ss = buf
        off = pl.multiple_of(k * K, 8)
        for j in range(0, K, L):
            rc = rc_all[pl.ds(off + j, L)]
            i_v[pl.ds(j, L)] = lax.shift_right_logical(rc, 16)
        pltpu.async_copy(h_hbm.at[i_v], rows, sg)

    def wait_scatter(buf):
        rows, i_v, sg, ss = buf
        pltpu.make_async_copy(rows, acc_sh.at[i_v], ss).wait()

    def body(k, buf):
        rows, i_v, sg, ss = buf
        pltpu.make_async_copy(h_hbm.at[i_v], rows, sg).wait()
        off = pl.multiple_of(k * K, 8)
        for j in range(0, K, L):
            rc = rc_all[pl.ds(off + j, L)]
            i_v[pl.ds(j, L)] = lax.bitwise_and(rc, jnp.int32(0xFFFF))

        @pl.loop(0, K, step=L)
        def _(j0):
            wv = w_all[pl.ds(off + j0, L)]
            for t in range(L):
                s = wv[t]
                for q in range(D // L):
                    rows[j0 + t, pl.ds(q * L, L)] = rows[j0 + t, pl.ds(q * L, L)] * s

        pltpu.async_copy(rows, acc_sh.at[i_v], ss, add=True)

    for d in range(DEPTH):
        start(d, bufs[d])

    @pl.loop(0, _LOOPED, step=NBUF)
    def _(kk):
        for b in range(NBUF):
            k = kk + b
            bn = (b + DEPTH) % NBUF

            @pl.when(k >= DEPTH)
            def _():
                wait_scatter(bufs[bn])

            start(k + DEPTH, bufs[bn])
            body(k, bufs[b])

    for k in range(_LOOPED, NCH):
        bn = (k + DEPTH) % NBUF
        if k >= DEPTH:
            wait_scatter(bufs[bn])
        if k + DEPTH < NCH:
            start(k + DEPTH, bufs[bn])
        body(k, bufs[k % NBUF])
    for d in range(DEPTH):
        wait_scatter(bufs[(NCH - DEPTH + d) % NBUF])

    toff = NCH * K
    rc = rc_all[pl.ds(toff, L)]
    i_t[pl.ds(0, L)] = lax.shift_right_logical(rc, 16)
    pltpu.async_copy(h_hbm.at[i_t], rows_t, sg0).wait()
    i_t[pl.ds(0, L)] = lax.bitwise_and(rc, jnp.int32(0xFFFF))
    wv = w_all[pl.ds(toff, L)]
    for t in range(L):
        s = wv[t]
        for q in range(D // L):
            rows_t[t, pl.ds(q * L, L)] = rows_t[t, pl.ds(q * L, L)] * s
    pltpu.sync_copy(rows_t, acc_sh.at[i_t], add=True)

    plsc.subcore_barrier()
    d0 = pl.multiple_of(cid * NPAD + sid * RPT, 8)
    _per_tile_rows(sid, lambda s, n: _drain_rows(acc_sh, out_hbm, rows0, s, d0, n))


@functools.partial(
    pl.kernel,
    out_type=jax.ShapeDtypeStruct((NC * NPAD,), jnp.float32),
    mesh=_MESH,
    compiler_params=pltpu.CompilerParams(needs_layout_passes=False),
    scratch_types=[
        pltpu.VMEM_SHARED((N,), jnp.float32),
        pltpu.VMEM((EPT,), jnp.int32),
        pltpu.VMEM((EPT,), jnp.int32),
        pltpu.VMEM((EPT,), jnp.float32),
        pltpu.VMEM((EPT,), jnp.float32),
        pltpu.VMEM((N,), jnp.float32),
        pltpu.VMEM((RPT,), jnp.float32),
        pltpu.SemaphoreType.DMA,
    ],
)
def _sc_agg1(g_hbm, r_hbm, c_hbm, w_hbm, out_hbm, acc_sh,
             r_v, c_v, w_v, g_v, h4_all, z_v, sem):
    cid, sid, base = _worker(EPT)
    _fill_zero(z_v)
    _per_tile_rows(sid, lambda s, n: _zero_rows(z_v, acc_sh, s, n))
    plsc.subcore_barrier()
    pltpu.sync_copy(r_hbm.at[pl.ds(base, EPT)], r_v)
    pltpu.sync_copy(c_hbm.at[pl.ds(base, EPT)], c_v)
    pltpu.sync_copy(w_hbm.at[pl.ds(base, EPT)], w_v)
    pltpu.sync_copy(g_hbm, h4_all)

    @pl.loop(0, EPT, step=L)
    def _(i):
        idx = r_v[pl.ds(i, L)]
        vals = plsc.load_gather(h4_all, [idx])
        g_v[pl.ds(i, L)] = vals * w_v[pl.ds(i, L)]

    pltpu.sync_copy(g_v, acc_sh.at[c_v], add=True)
    plsc.subcore_barrier()
    d0 = pl.multiple_of(cid * NPAD + sid * RPT, 8)
    _per_tile_rows(sid, lambda s, n: _drain_rows(acc_sh, out_hbm, z_v, s, d0, n))


_BLK = 1000
_GRID = N // _BLK


def _dis_body(deg_ref, dis_ref):
    d = deg_ref[0, :] + deg_ref[1, :] + 1.0
    dis_ref[...] = lax.rsqrt(d)


def _tc_dis(deg_parts):
    return pl.pallas_call(
        _dis_body,
        out_shape=jax.ShapeDtypeStruct((NPAD,), jnp.float32),
    )(deg_parts)


def _mm2_body(x_ref, wa_ref, wb_ref, dis_ref, hp_ref, hb_ref):
    x = x_ref[...]
    d = dis_ref[...]
    ha = lax.dot_general(x, wa_ref[...], (((1,), (1,)), ((), ())),
                         precision=_PREC, preferred_element_type=jnp.float32)
    hp_ref[...] = d * ha
    hb_ref[...] = lax.dot_general(x, wb_ref[...], (((1,), (1,)), ((), ())),
                                  precision=_PREC,
                                  preferred_element_type=jnp.float32)


def _tc_mm2(x, wa, wb, dis):
    return pl.pallas_call(
        _mm2_body,
        grid=(_GRID,),
        in_specs=[
            pl.BlockSpec((_BLK, D), lambda i: (i, 0)),
            pl.BlockSpec((D, D), lambda i: (0, 0)),
            pl.BlockSpec((D, D), lambda i: (0, 0)),
            pl.BlockSpec((_BLK, 1), lambda i: (i, 0)),
        ],
        out_specs=[
            pl.BlockSpec((_BLK, D), lambda i: (i, 0)),
            pl.BlockSpec((_BLK, D), lambda i: (i, 0)),
        ],
        out_shape=[
            jax.ShapeDtypeStruct((N, D), jnp.float32),
            jax.ShapeDtypeStruct((N, D), jnp.float32),
        ],
    )(x, wa, wb, dis)


def _post_mm_body(s_ref, h_ref, dis_ref, b_ref, w_ref, out_ref):
    d = dis_ref[...]
    z = jnp.maximum(d * (s_ref[0] + s_ref[1] + h_ref[...]) + b_ref[...], 0.0)
    o = lax.dot_general(z, w_ref[...], (((1,), (1,)), ((), ())),
                        precision=_PREC, preferred_element_type=jnp.float32)
    out_ref[...] = d * o


def _tc_post_mm(s_parts, hp, dis, b, w, d_out):
    return pl.pallas_call(
        _post_mm_body,
        grid=(_GRID,),
        in_specs=[
            pl.BlockSpec((2, _BLK, D), lambda i: (0, i, 0)),
            pl.BlockSpec((_BLK, D), lambda i: (i, 0)),
            pl.BlockSpec((_BLK, 1), lambda i: (i, 0)),
            pl.BlockSpec((1, D), lambda i: (0, 0)),
            pl.BlockSpec((d_out, D), lambda i: (0, 0)),
        ],
        out_specs=pl.BlockSpec((_BLK, d_out), lambda i: (i, 0)),
        out_shape=jax.ShapeDtypeStruct((N, d_out), jnp.float32),
    )(s_parts, hp, dis, b, w)


def _xsol_body(s_ref, h_ref, dis_ref, b_ref, wol_ref, bol_ref, hga_ref,
               wg1b_ref, xsol_ref, hgp_ref):
    d = dis_ref[...]
    z = d * (s_ref[0] + s_ref[1] + h_ref[...]) + b_ref[...]
    xsol = lax.dot_general(z, wol_ref[...], (((1,), (1,)), ((), ())),
                           precision=_PREC,
                           preferred_element_type=jnp.float32) + bol_ref[...]
    xsol_ref[...] = xsol
    hgp_ref[...] = d * (hga_ref[...] + xsol[:, D - 1:D] * wg1b_ref[...])


def _tc_xsol(s_parts, hp, dis, b, wol, bol, hga, wg1b):
    return pl.pallas_call(
        _xsol_body,
        grid=(_GRID,),
        in_specs=[
            pl.BlockSpec((2, _BLK, D), lambda i: (0, i, 0)),
            pl.BlockSpec((_BLK, D), lambda i: (i, 0)),
            pl.BlockSpec((_BLK, 1), lambda i: (i, 0)),
            pl.BlockSpec((1, D), lambda i: (0, 0)),
            pl.BlockSpec((D, D), lambda i: (0, 0)),
            pl.BlockSpec((1, D), lambda i: (0, 0)),
            pl.BlockSpec((_BLK, D), lambda i: (i, 0)),
            pl.BlockSpec((1, D), lambda i: (0, 0)),
        ],
        out_specs=[
            pl.BlockSpec((_BLK, D), lambda i: (i, 0)),
            pl.BlockSpec((_BLK, D), lambda i: (i, 0)),
        ],
        out_shape=[
            jax.ShapeDtypeStruct((N, D), jnp.float32),
            jax.ShapeDtypeStruct((N, D), jnp.float32),
        ],
    )(s_parts, hp, dis, b, wol, bol, hga, wg1b)


def _final_body(s_ref, h4p_ref, dis_ref, consts_ref, xsol_ref, xlast_ref,
                xnew_ref, gamma_ref):
    d = dis_ref[...]
    bg2 = consts_ref[0, 0]
    wgl = consts_ref[0, 1]
    bgl = consts_ref[0, 2]
    g = d * (s_ref[0] + s_ref[1] + h4p_ref[...]) + bg2
    gl = g * wgl + bgl
    gmin = jnp.min(gl)
    gmax = jnp.max(gl)
    gamma = (gl - gmin) / (gmax - gmin + 1e-12)
    gamma_ref[...] = gamma
    xsol = xsol_ref[...]
    xl = xlast_ref[...]
    upd = xl + gamma * (xsol[:, D - 1:D] - xl)
    col = lax.broadcasted_iota(jnp.int32, (N, D), 1)
    xnew_ref[...] = jnp.where(col == D - 1, upd, xsol)


def _tc_final(s_parts, h4p, dis, consts, xsol, xlast):
    return pl.pallas_call(
        _final_body,
        out_shape=[
            jax.ShapeDtypeStruct((N, D), jnp.float32),
            jax.ShapeDtypeStruct((N, 1), jnp.float32),
        ],
    )(s_parts, h4p, dis, consts, xsol, xlast)


def kernel(x, edge_index, edge_weights, Wo1, bo1, Wo2, bo2, Wol, bol,
           Wg1, bg1, Wg2, bg2, Wgl, bgl):
    row = edge_index[0].astype(jnp.int32)
    col = edge_index[1].astype(jnp.int32)
    ew = edge_weights.astype(jnp.float32)

    wg1a = Wg1[:, :D]
    wg1b = Wg1[:, D].reshape(1, D)
    rc = jnp.bitwise_or(jnp.left_shift(row, 16), col)

    deg_parts = _sc_deg(col, ew).reshape(NC, NPAD)
    dis1 = _tc_dis(deg_parts)[:N]
    dis = dis1.reshape(N, 1)

    h1p, hga = _tc_mm2(x, Wo1, wg1a, dis)

    s1 = _sc_agg(h1p, rc, ew).reshape(NC, NPAD, D)
    h2p = _tc_post_mm(s1, h1p, dis, bo1.reshape(1, D), Wo2, D)

    s2 = _sc_agg(h2p, rc, ew).reshape(NC, NPAD, D)
    xsol, hgp = _tc_xsol(s2, h2p, dis, bo2.reshape(1, D), Wol,
                         bol.reshape(1, D), hga, wg1b)

    s3 = _sc_agg(hgp, rc, ew).reshape(NC, NPAD, D)
    h4p = _tc_post_mm(s3, hgp, dis, bg1.reshape(1, D), Wg2, 1)

    s4 = _sc_agg1(h4p.reshape(N), row, col, ew).reshape(NC, NPAD)[:, :N]

    consts = jnp.stack([bg2[0], Wgl[0, 0], bgl[0]]).reshape(1, 3)
    xnew, gamma = _tc_final(s4.reshape(NC, N, 1), h4p, dis, consts,
                            xsol, x[:, D - 1:D])
    return (xnew, gamma)


# speedup vs baseline: 1.4988x; 1.0052x over previous
import functools

import jax
import jax.numpy as jnp
from jax import lax
from jax.experimental import pallas as pl
from jax.experimental.pallas import tpu as pltpu
from jax.experimental.pallas import tpu_sc as plsc

N = 10000
E = 320000
D = 128
NC, NS, L = 2, 16, 16
NW = NC * NS
EPT = E // NW
RPT = 632
RPT_LAST = N - 15 * RPT
NPAD = RPT * NS
K = 32
NCH = EPT // K
TAIL = EPT - NCH * K
NBUF = 6
DEPTH = NBUF // 2
_LOOPED = (NCH - DEPTH) - (NCH - DEPTH) % NBUF

_MESH = plsc.VectorSubcoreMesh(core_axis_name="c", subcore_axis_name="s")
_PREC = lax.Precision.HIGHEST


def _worker(base_len):
    cid = lax.axis_index("c")
    sid = lax.axis_index("s")
    wid = sid * NC + cid
    return cid, sid, pl.multiple_of(wid * base_len, 8)


def _zero_rows(z_v, acc_sh, start, nrows):
    nz = z_v.shape[0]
    off = 0
    while off < nrows:
        step = min(nz, nrows - off)
        pltpu.sync_copy(z_v.at[pl.ds(0, step)], acc_sh.at[pl.ds(start + off, step)])
        off += step


def _fill_zero(z_v):
    nz = z_v.shape[0]
    zero = jnp.zeros((L,), jnp.float32)

    if len(z_v.shape) == 1:
        @pl.loop(0, nz, step=L)
        def _(i):
            z_v[pl.ds(i, L)] = zero
    else:
        @pl.loop(0, nz)
        def _(j):
            for q in range(z_v.shape[1] // L):
                z_v[j, pl.ds(q * L, L)] = zero


def _drain_rows(acc_sh, out_hbm, stage, start, dst, nrows):
    off = 0
    nz = stage.shape[0]
    while off < nrows:
        step = min(nz, nrows - off)
        pltpu.sync_copy(acc_sh.at[pl.ds(start + off, step)], stage.at[pl.ds(0, step)])
        pltpu.sync_copy(stage.at[pl.ds(0, step)], out_hbm.at[pl.ds(dst + off, step)])
        off += step


def _per_tile_rows(sid, fn):
    start = pl.multiple_of(sid * RPT, 8)

    @pl.when(sid < NS - 1)
    def _():
        fn(start, RPT)

    @pl.when(sid == NS - 1)
    def _():
        fn(start, RPT_LAST)


@functools.partial(
    pl.kernel,
    out_type=jax.ShapeDtypeStruct((NC * NPAD,), jnp.float32),
    mesh=_MESH,
    scratch_types=[
        pltpu.VMEM_SHARED((N,), jnp.float32),
        pltpu.VMEM((EPT,), jnp.float32),
        pltpu.VMEM((EPT,), jnp.int32),
        pltpu.VMEM((RPT,), jnp.float32),
    ],
)
def _sc_deg(c_hbm, w_hbm, out_hbm, acc_sh, w_v, c_v, z_v):
    cid, sid, base = _worker(EPT)
    _fill_zero(z_v)
    _per_tile_rows(sid, lambda s, n: _zero_rows(z_v, acc_sh, s, n))
    plsc.subcore_barrier()
    pltpu.sync_copy(w_hbm.at[pl.ds(base, EPT)], w_v)
    pltpu.sync_copy(c_hbm.at[pl.ds(base, EPT)], c_v)
    pltpu.sync_copy(w_v, acc_sh.at[c_v], add=True)
    plsc.subcore_barrier()
    d0 = pl.multiple_of(cid * NPAD + sid * RPT, 8)
    _per_tile_rows(sid, lambda s, n: _drain_rows(acc_sh, out_hbm, z_v, s, d0, n))


@functools.partial(
    pl.kernel,
    out_type=jax.ShapeDtypeStruct((NC * NPAD, D), jnp.float32),
    mesh=_MESH,
    scratch_types=[
        pltpu.VMEM_SHARED((N, D), jnp.float32),
        pltpu.VMEM((K, D), jnp.float32),
        pltpu.VMEM((K, D), jnp.float32),
        pltpu.VMEM((K, D), jnp.float32),
        pltpu.VMEM((K, D), jnp.float32),
        pltpu.VMEM((K, D), jnp.float32),
        pltpu.VMEM((K, D), jnp.float32),
        pltpu.VMEM((TAIL, D), jnp.float32),
        pltpu.VMEM((EPT,), jnp.int32),
        pltpu.VMEM((EPT,), jnp.float32),
        pltpu.VMEM((K,), jnp.int32),
        pltpu.VMEM((K,), jnp.int32),
        pltpu.VMEM((K,), jnp.int32),
        pltpu.VMEM((K,), jnp.int32),
        pltpu.VMEM((K,), jnp.int32),
        pltpu.VMEM((K,), jnp.int32),
        pltpu.VMEM((TAIL,), jnp.int32),
        pltpu.SemaphoreType.DMA,
        pltpu.SemaphoreType.DMA,
        pltpu.SemaphoreType.DMA,
        pltpu.SemaphoreType.DMA,
        pltpu.SemaphoreType.DMA,
        pltpu.SemaphoreType.DMA,
        pltpu.SemaphoreType.DMA,
        pltpu.SemaphoreType.DMA,
        pltpu.SemaphoreType.DMA,
        pltpu.SemaphoreType.DMA,
        pltpu.SemaphoreType.DMA,
        pltpu.SemaphoreType.DMA,
    ],
)
def _sc_agg(h_hbm, rc_hbm, w_hbm, out_hbm, acc_sh,
            rows0, rows1, rows2, rows3, rows4, rows5, rows_t, rc_all, w_all,
            i0, i1, i2, i3, i4, i5, i_t,
            sg0, sg1, sg2, sg3, sg4, sg5, ss0, ss1, ss2, ss3, ss4, ss5):
    cid, sid, base = _worker(EPT)
    _fill_zero(rows0)
    _per_tile_rows(sid, lambda s, n: _zero_rows(rows0, acc_sh, s, n))
    pltpu.sync_copy(rc_hbm.at[pl.ds(base, EPT)], rc_all)
    pltpu.sync_copy(w_hbm.at[pl.ds(base, EPT)], w_all)
    plsc.subcore_barrier()

    bufs = ((rows0, i0, sg0, ss0), (rows1, i1, sg1, ss1),
            (rows2, i2, sg2, ss2), (rows3, i3, sg3, ss3),
            (rows4, i4, sg4, ss4), (rows5, i5, sg5, ss5))

    def start(k, buf):
        rows, i_v, sg, ss = buf
        off = pl.multiple_of(k * K, 8)
        for j in range(0, K, L):
            rc = rc_all[pl.ds(off + j, L)]
            i_v[pl.ds(j, L)] = lax.shift_right_logical(rc, 16)
        pltpu.async_copy(h_hbm.at[i_v], rows, sg)

    def wait_scatter(buf):
        rows, i_v, sg, ss = buf
        pltpu.make_async_copy(rows, acc_sh.at[i_v], ss).wait()

    def body(k, buf):
        rows, i_v, sg, ss = buf
        pltpu.make_async_copy(h_hbm.at[i_v], rows, sg).wait()
        off = pl.multiple_of(k * K, 8)
        for j in range(0, K, L):
            rc = rc_all[pl.ds(off + j, L)]
            i_v[pl.ds(j, L)] = lax.bitwise_and(rc, jnp.int32(0xFFFF))

        @pl.loop(0, K, step=L)
        def _(j0):
            wv = w_all[pl.ds(off + j0, L)]
            for t in range(L):
                s = wv[t]
                for q in range(D // L):
                    rows[j0 + t, pl.ds(q * L, L)] = rows[j0 + t, pl.ds(q * L, L)] * s

        pltpu.async_copy(rows, acc_sh.at[i_v], ss, add=True)

    for d in range(DEPTH):
        start(d, bufs[d])

    @pl.loop(0, _LOOPED, step=NBUF)
    def _(kk):
        for b in range(NBUF):
            k = kk + b
            bn = (b + DEPTH) % NBUF

            @pl.when(k >= DEPTH)
            def _():
                wait_scatter(bufs[bn])

            start(k + DEPTH, bufs[bn])
            body(k, bufs[b])

    for k in range(_LOOPED, NCH):
        bn = (k + DEPTH) % NBUF
        if k >= DEPTH:
            wait_scatter(bufs[bn])
        if k + DEPTH < NCH:
            start(k + DEPTH, bufs[bn])
        body(k, bufs[k % NBUF])
    for d in range(DEPTH):
        wait_scatter(bufs[(NCH - DEPTH + d) % NBUF])

    toff = NCH * K
    rc = rc_all[pl.ds(toff, L)]
    i_t[pl.ds(0, L)] = lax.shift_right_logical(rc, 16)
    pltpu.async_copy(h_hbm.at[i_t], rows_t, sg0).wait()
    i_t[pl.ds(0, L)] = lax.bitwise_and(rc, jnp.int32(0xFFFF))
    wv = w_all[pl.ds(toff, L)]
    for t in range(L):
        s = wv[t]
        for q in range(D // L):
            rows_t[t, pl.ds(q * L, L)] = rows_t[t, pl.ds(q * L, L)] * s
    pltpu.sync_copy(rows_t, acc_sh.at[i_t], add=True)

    plsc.subcore_barrier()
    d0 = pl.multiple_of(cid * NPAD + sid * RPT, 8)
    _per_tile_rows(sid, lambda s, n: _drain_rows(acc_sh, out_hbm, rows0, s, d0, n))


@functools.partial(
    pl.kernel,
    out_type=jax.ShapeDtypeStruct((NC * NPAD,), jnp.float32),
    mesh=_MESH,
    compiler_params=pltpu.CompilerParams(needs_layout_passes=False),
    scratch_types=[
        pltpu.VMEM_SHARED((N,), jnp.float32),
        pltpu.VMEM((EPT,), jnp.int32),
        pltpu.VMEM((EPT,), jnp.int32),
        pltpu.VMEM((EPT,), jnp.float32),
        pltpu.VMEM((EPT,), jnp.float32),
        pltpu.VMEM((N,), jnp.float32),
        pltpu.VMEM((RPT,), jnp.float32),
        pltpu.SemaphoreType.DMA,
    ],
)
def _sc_agg1(g_hbm, r_hbm, c_hbm, w_hbm, out_hbm, acc_sh,
             r_v, c_v, w_v, g_v, h4_all, z_v, sem):
    cid, sid, base = _worker(EPT)
    _fill_zero(z_v)
    _per_tile_rows(sid, lambda s, n: _zero_rows(z_v, acc_sh, s, n))
    plsc.subcore_barrier()
    pltpu.sync_copy(r_hbm.at[pl.ds(base, EPT)], r_v)
    pltpu.sync_copy(c_hbm.at[pl.ds(base, EPT)], c_v)
    pltpu.sync_copy(w_hbm.at[pl.ds(base, EPT)], w_v)
    pltpu.sync_copy(g_hbm, h4_all)

    @pl.loop(0, EPT, step=L)
    def _(i):
        idx = r_v[pl.ds(i, L)]
        vals = plsc.load_gather(h4_all, [idx])
        g_v[pl.ds(i, L)] = vals * w_v[pl.ds(i, L)]

    pltpu.sync_copy(g_v, acc_sh.at[c_v], add=True)
    plsc.subcore_barrier()
    d0 = pl.multiple_of(cid * NPAD + sid * RPT, 8)
    _per_tile_rows(sid, lambda s, n: _drain_rows(acc_sh, out_hbm, z_v, s, d0, n))


_BLK = 1000
_GRID = N // _BLK


def _dis_body(deg_ref, dis_ref):
    d = deg_ref[0, :] + deg_ref[1, :] + 1.0
    dis_ref[...] = lax.rsqrt(d).reshape(NPAD, 1)


def _tc_dis(deg_parts):
    return pl.pallas_call(
        _dis_body,
        out_shape=jax.ShapeDtypeStruct((NPAD, 1), jnp.float32),
    )(deg_parts)


def _mm2_body(x_ref, wa_ref, wb_ref, dis_ref, hp_ref, hb_ref):
    x = x_ref[...]
    d = dis_ref[...]
    ha = lax.dot_general(x, wa_ref[...], (((1,), (1,)), ((), ())),
                         precision=_PREC, preferred_element_type=jnp.float32)
    hp_ref[...] = d * ha
    hb_ref[...] = lax.dot_general(x, wb_ref[...], (((1,), (1,)), ((), ())),
                                  precision=_PREC,
                                  preferred_element_type=jnp.float32)


def _tc_mm2(x, wa, wb, dis):
    return pl.pallas_call(
        _mm2_body,
        grid=(_GRID,),
        in_specs=[
            pl.BlockSpec((_BLK, D), lambda i: (i, 0)),
            pl.BlockSpec((D, D), lambda i: (0, 0)),
            pl.BlockSpec((D, D), lambda i: (0, 0)),
            pl.BlockSpec((_BLK, 1), lambda i: (i, 0)),
        ],
        out_specs=[
            pl.BlockSpec((_BLK, D), lambda i: (i, 0)),
            pl.BlockSpec((_BLK, D), lambda i: (i, 0)),
        ],
        out_shape=[
            jax.ShapeDtypeStruct((N, D), jnp.float32),
            jax.ShapeDtypeStruct((N, D), jnp.float32),
        ],
    )(x, wa, wb, dis)


def _post_mm_body(s_ref, h_ref, dis_ref, b_ref, w_ref, out_ref):
    d = dis_ref[...]
    z = jnp.maximum(d * (s_ref[0] + s_ref[1] + h_ref[...]) + b_ref[...], 0.0)
    o = lax.dot_general(z, w_ref[...], (((1,), (1,)), ((), ())),
                        precision=_PREC, preferred_element_type=jnp.float32)
    out_ref[...] = d * o


def _tc_post_mm(s_parts, hp, dis, b, w, d_out):
    return pl.pallas_call(
        _post_mm_body,
        grid=(_GRID,),
        in_specs=[
            pl.BlockSpec((2, _BLK, D), lambda i: (0, i, 0)),
            pl.BlockSpec((_BLK, D), lambda i: (i, 0)),
            pl.BlockSpec((_BLK, 1), lambda i: (i, 0)),
            pl.BlockSpec((1, D), lambda i: (0, 0)),
            pl.BlockSpec((d_out, D), lambda i: (0, 0)),
        ],
        out_specs=pl.BlockSpec((_BLK, d_out), lambda i: (i, 0)),
        out_shape=jax.ShapeDtypeStruct((N, d_out), jnp.float32),
    )(s_parts, hp, dis, b, w)


def _xsol_body(s_ref, h_ref, dis_ref, b_ref, wol_ref, bol_ref, hga_ref,
               wg1b_ref, xsol_ref, hgp_ref):
    d = dis_ref[...]
    z = d * (s_ref[0] + s_ref[1] + h_ref[...]) + b_ref[...]
    xsol = lax.dot_general(z, wol_ref[...], (((1,), (1,)), ((), ())),
                           precision=_PREC,
                           preferred_element_type=jnp.float32) + bol_ref[...]
    xsol_ref[...] = xsol
    hgp_ref[...] = d * (hga_ref[...] + xsol[:, D - 1:D] * wg1b_ref[...])


def _tc_xsol(s_parts, hp, dis, b, wol, bol, hga, wg1b):
    return pl.pallas_call(
        _xsol_body,
        grid=(_GRID,),
        in_specs=[
            pl.BlockSpec((2, _BLK, D), lambda i: (0, i, 0)),
            pl.BlockSpec((_BLK, D), lambda i: (i, 0)),
            pl.BlockSpec((_BLK, 1), lambda i: (i, 0)),
            pl.BlockSpec((1, D), lambda i: (0, 0)),
            pl.BlockSpec((D, D), lambda i: (0, 0)),
            pl.BlockSpec((1, D), lambda i: (0, 0)),
            pl.BlockSpec((_BLK, D), lambda i: (i, 0)),
            pl.BlockSpec((1, D), lambda i: (0, 0)),
        ],
        out_specs=[
            pl.BlockSpec((_BLK, D), lambda i: (i, 0)),
            pl.BlockSpec((_BLK, D), lambda i: (i, 0)),
        ],
        out_shape=[
            jax.ShapeDtypeStruct((N, D), jnp.float32),
            jax.ShapeDtypeStruct((N, D), jnp.float32),
        ],
    )(s_parts, hp, dis, b, wol, bol, hga, wg1b)


def _final_body(s_ref, h4p_ref, dis_ref, consts_ref, xsol_ref, xlast_ref,
                xnew_ref, gamma_ref):
    d = dis_ref[...]
    bg2 = consts_ref[0, 0]
    wgl = consts_ref[0, 1]
    bgl = consts_ref[0, 2]
    g = d * (s_ref[0] + s_ref[1] + h4p_ref[...]) + bg2
    gl = g * wgl + bgl
    gmin = jnp.min(gl)
    gmax = jnp.max(gl)
    gamma = (gl - gmin) / (gmax - gmin + 1e-12)
    gamma_ref[...] = gamma
    xsol = xsol_ref[...]
    xl = xlast_ref[...]
    upd = xl + gamma * (xsol[:, D - 1:D] - xl)
    col = lax.broadcasted_iota(jnp.int32, (N, D), 1)
    xnew_ref[...] = jnp.where(col == D - 1, upd, xsol)


def _tc_final(s_parts, h4p, dis, consts, xsol, xlast):
    return pl.pallas_call(
        _final_body,
        grid=(1,),
        in_specs=[
            pl.BlockSpec((2, N, 1), lambda i: (0, 0, 0)),
            pl.BlockSpec((N, 1), lambda i: (0, 0)),
            pl.BlockSpec((N, 1), lambda i: (0, 0)),
            pl.BlockSpec((1, 3), lambda i: (0, 0)),
            pl.BlockSpec((N, D), lambda i: (0, 0)),
            pl.BlockSpec((N, 1), lambda i: (0, 0)),
        ],
        out_specs=[
            pl.BlockSpec((N, D), lambda i: (0, 0)),
            pl.BlockSpec((N, 1), lambda i: (0, 0)),
        ],
        out_shape=[
            jax.ShapeDtypeStruct((N, D), jnp.float32),
            jax.ShapeDtypeStruct((N, 1), jnp.float32),
        ],
    )(s_parts, h4p, dis, consts, xsol, xlast)


def kernel(x, edge_index, edge_weights, Wo1, bo1, Wo2, bo2, Wol, bol,
           Wg1, bg1, Wg2, bg2, Wgl, bgl):
    row = edge_index[0].astype(jnp.int32)
    col = edge_index[1].astype(jnp.int32)
    ew = edge_weights.astype(jnp.float32)

    wg1a = Wg1[:, :D]
    wg1b = Wg1[:, D].reshape(1, D)
    rc = jnp.bitwise_or(jnp.left_shift(row, 16), col)

    deg_parts = _sc_deg(col, ew).reshape(NC, NPAD)
    dis = _tc_dis(deg_parts)

    h1p, hga = _tc_mm2(x, Wo1, wg1a, dis)

    s1 = _sc_agg(h1p, rc, ew).reshape(NC, NPAD, D)
    h2p = _tc_post_mm(s1, h1p, dis, bo1.reshape(1, D), Wo2, D)

    s2 = _sc_agg(h2p, rc, ew).reshape(NC, NPAD, D)
    xsol, hgp = _tc_xsol(s2, h2p, dis, bo2.reshape(1, D), Wol,
                         bol.reshape(1, D), hga, wg1b)

    s3 = _sc_agg(hgp, rc, ew).reshape(NC, NPAD, D)
    h4p = _tc_post_mm(s3, hgp, dis, bg1.reshape(1, D), Wg2, 1)

    s4 = _sc_agg1(h4p.reshape(N), row, col, ew).reshape(NC, NPAD)[:, :N]

    consts = jnp.stack([bg2[0], Wgl[0, 0], bgl[0]]).reshape(1, 3)
    xnew, gamma = _tc_final(s4.reshape(NC, N, 1), h4p, dis, consts,
                            xsol, x[:, D - 1:D])
    return (xnew, gamma)
